# Initial kernel scaffold; baseline (speedup 1.0000x reference)
import jax
import jax.numpy as jnp
from jax.experimental import pallas as pl


def kernel(x_ggnn, x_appnp, edge_index, batch, W1, b1, gamma, beta, W2, b2, W3, b3):
    raise NotImplementedError("write your pallas kernel here")


# speedup vs baseline: 56.8174x; 56.8174x over previous
import functools
import math

import jax
import jax.numpy as jnp
from jax import lax
from jax.experimental import pallas as pl
from jax.experimental.pallas import tpu as pltpu
from jax.experimental.pallas import tpu_sc as plsc

_N = 10000
_E = 320000
_B = 64
_H = 128
_MAX_NODES = 500
_F32 = jnp.float32

_DN_T = (((0,), (0,)), ((), ()))
_DN_M = (((1,), (0,)), ((), ()))
_PREC = jax.lax.Precision.HIGHEST


def _dot(a, b, dn):
    return jax.lax.dot_general(a, b, dn, precision=_PREC,
                               preferred_element_type=_F32)


def _tc_body(ne_ref, batch_ref, xg_ref, xa_ref, W1_ref, b1_ref, g_ref,
             be_ref, W2_ref, b2_ref, W3l_ref, W3g_ref, b3l_ref, b3g_ref,
             out_ref, wl_ref, wg_ref, cf_ref):
    xg = xg_ref[...]
    xa = xa_ref[...]
    bat = batch_ref[...]
    lanes = jax.lax.broadcasted_iota(jnp.int32, (1, _B), 1)
    onehot = (bat == lanes).astype(_F32)

    ones_n = jnp.full((_N, 1), 1.0, _F32)
    counts = _dot(onehot, ones_n, _DN_T)
    sum_g = _dot(onehot, xg, _DN_T)
    sum_a = _dot(onehot, xa, _DN_T)
    pooled_g = sum_g / counts
    pooled_a = sum_a / counts

    nw = ne_ref.shape[0]
    ne = _dot(ne_ref[...], jnp.full((nw, 1), 1.0, _F32), _DN_T)

    n_nodes = counts
    scale = jnp.log(n_nodes + 1.0) * (1.0 / math.log(_MAX_NODES + 1))
    density = ne / (n_nodes * (n_nodes - 1.0) + 1e-08)
    avg_degree = ne / (n_nodes + 1e-08)
    adn = jnp.minimum(avg_degree / 10.0, 1.0)

    num = jnp.sum(pooled_g * pooled_a, axis=1, keepdims=True)
    ng = jnp.sqrt(jnp.sum(pooled_g * pooled_g, axis=1, keepdims=True))
    na = jnp.sqrt(jnp.sum(pooled_a * pooled_a, axis=1, keepdims=True))
    cos = num / (jnp.maximum(ng, 1e-08) * jnp.maximum(na, 1e-08))
    div = (1.0 - cos) * 0.5

    lane4 = jax.lax.broadcasted_iota(jnp.int32, (_B, 4), 1)
    cf = (jnp.where(lane4 == 0, scale, 0.0)
          + jnp.where(lane4 == 1, density, 0.0)
          + jnp.where(lane4 == 2, adn, 0.0)
          + jnp.where(lane4 == 3, div, 0.0))
    cf_ref[...] = cf

    h = (_dot(pooled_g, W1_ref[0:_H, :], _DN_M)
         + _dot(pooled_a, W1_ref[_H:2 * _H, :], _DN_M)
         + _dot(cf, W1_ref[2 * _H:2 * _H + 4, :], _DN_M)
         + b1_ref[...])
    mu = jnp.mean(h, axis=1, keepdims=True)
    var = jnp.mean((h - mu) ** 2, axis=1, keepdims=True)
    h = (h - mu) * jax.lax.rsqrt(var + 1e-05) * g_ref[...] + be_ref[...]
    h = jnp.maximum(h, 0.0)
    h2 = jnp.maximum(_dot(h, W2_ref[...], _DN_M) + b2_ref[...], 0.0)
    raw_l = _dot(h2, W3l_ref[...], _DN_M) + b3l_ref[...]
    raw_g = _dot(h2, W3g_ref[...], _DN_M) + b3g_ref[...]
    m = jnp.maximum(raw_l, raw_g)
    el = jnp.exp(raw_l - m)
    eg = jnp.exp(raw_g - m)
    s = el + eg
    wl = el / s
    wg = eg / s
    wl_ref[...] = wl
    wg_ref[...] = wg

    wn_l = _dot(onehot, wl, _DN_M)
    wn_g = _dot(onehot, wg, _DN_M)
    out_ref[...] = wn_l * xg + wn_g * xa


def _sc_edge_hist(src, dst, batch):
    info = plsc.get_sparse_core_info()
    nc, ns, nl = info.num_cores, info.num_subcores, info.num_lanes
    nw = nc * ns
    epw = _E // nw
    mesh = plsc.VectorSubcoreMesh(core_axis_name="c", subcore_axis_name="s")

    @functools.partial(
        pl.kernel, mesh=mesh,
        compiler_params=pltpu.CompilerParams(needs_layout_passes=False),
        out_type=jax.ShapeDtypeStruct((nw, _B), _F32),
        scratch_types=[
            pltpu.VMEM((_N,), jnp.int32),
            pltpu.VMEM((epw,), jnp.int32),
            pltpu.VMEM((epw,), jnp.int32),
            pltpu.VMEM((_B,), _F32),
        ],
    )
    def sc_kern(src_hbm, dst_hbm, batch_hbm, out_hbm, batch_v, src_v, dst_v,
                cnt_v):
        wid = lax.axis_index("s") * nc + lax.axis_index("c")
        base = wid * epw
        pltpu.sync_copy(batch_hbm, batch_v)
        pltpu.sync_copy(src_hbm.at[pl.ds(base, epw)], src_v)
        pltpu.sync_copy(dst_hbm.at[pl.ds(base, epw)], dst_v)
        zeros = jnp.zeros((nl,), _F32)
        for k in range(_B // nl):
            cnt_v[pl.ds(k * nl, nl)] = zeros
        ones = jnp.ones((nl,), _F32)

        def body(i, carry):
            sl = src_v[pl.ds(i * nl, nl)]
            dl = dst_v[pl.ds(i * nl, nl)]
            bs = plsc.load_gather(batch_v, [sl])
            bd = plsc.load_gather(batch_v, [dl])
            plsc.addupdate_scatter(cnt_v, [bs], ones, mask=bs == bd)
            return carry

        lax.fori_loop(0, epw // nl, body, 0)
        pltpu.sync_copy(cnt_v, out_hbm.at[wid])

    return sc_kern(src, dst, batch)


def kernel(x_ggnn, x_appnp, edge_index, batch, W1, b1, gamma, beta,
           W2, b2, W3, b3):
    ne32 = _sc_edge_hist(edge_index[0], edge_index[1], batch)

    batch2d = batch.reshape(_N, 1)
    b1_2d = b1.reshape(1, _H)
    gamma_2d = gamma.reshape(1, _H)
    beta_2d = beta.reshape(1, _H)
    b2_2d = b2.reshape(1, _H // 2)
    W3l = W3[:, 0:1]
    W3g = W3[:, 1:2]
    b3l = b3[0:1].reshape(1, 1)
    b3g = b3[1:2].reshape(1, 1)

    out, wl, wg, cf = pl.pallas_call(
        _tc_body,
        out_shape=[
            jax.ShapeDtypeStruct((_N, _H), _F32),
            jax.ShapeDtypeStruct((_B, 1), _F32),
            jax.ShapeDtypeStruct((_B, 1), _F32),
            jax.ShapeDtypeStruct((_B, 4), _F32),
        ],
    )(ne32, batch2d, x_ggnn, x_appnp, W1, b1_2d, gamma_2d, beta_2d,
      W2, b2_2d, W3l, W3g, b3l, b3g)

    return out, wl.reshape(_B), wg.reshape(_B), cf


# speedup vs baseline: 66.6296x; 1.1727x over previous
import functools
import math

import jax
import jax.numpy as jnp
from jax import lax
from jax.experimental import pallas as pl
from jax.experimental.pallas import tpu as pltpu
from jax.experimental.pallas import tpu_sc as plsc

_N = 10000
_E = 320000
_B = 64
_H = 128
_MAX_NODES = 500
_F32 = jnp.float32

_DN_T = (((0,), (0,)), ((), ()))
_DN_M = (((1,), (0,)), ((), ()))
_PREC = jax.lax.Precision.DEFAULT


def _dot(a, b, dn):
    return jax.lax.dot_general(a, b, dn, precision=_PREC,
                               preferred_element_type=_F32)


def _tc_body(ne_ref, batch_ref, xg_ref, xa_ref, W1_ref, b1_ref, g_ref,
             be_ref, W2_ref, b2_ref, W3l_ref, W3g_ref, b3l_ref, b3g_ref,
             out_ref, wl_ref, wg_ref, cf_ref):
    xg = xg_ref[...]
    xa = xa_ref[...]
    bat = batch_ref[...]
    lanes = jax.lax.broadcasted_iota(jnp.int32, (1, _B), 1)
    onehot = (bat == lanes).astype(_F32)

    ones_n = jnp.full((_N, 1), 1.0, _F32)
    counts = _dot(onehot, ones_n, _DN_T)
    sum_g = _dot(onehot, xg, _DN_T)
    sum_a = _dot(onehot, xa, _DN_T)
    pooled_g = sum_g / counts
    pooled_a = sum_a / counts

    nw = ne_ref.shape[0]
    ne = _dot(ne_ref[...], jnp.full((nw, 1), 1.0, _F32), _DN_T)

    n_nodes = counts
    scale = jnp.log(n_nodes + 1.0) * (1.0 / math.log(_MAX_NODES + 1))
    density = ne / (n_nodes * (n_nodes - 1.0) + 1e-08)
    avg_degree = ne / (n_nodes + 1e-08)
    adn = jnp.minimum(avg_degree / 10.0, 1.0)

    num = jnp.sum(pooled_g * pooled_a, axis=1, keepdims=True)
    ng = jnp.sqrt(jnp.sum(pooled_g * pooled_g, axis=1, keepdims=True))
    na = jnp.sqrt(jnp.sum(pooled_a * pooled_a, axis=1, keepdims=True))
    cos = num / (jnp.maximum(ng, 1e-08) * jnp.maximum(na, 1e-08))
    div = (1.0 - cos) * 0.5

    lane4 = jax.lax.broadcasted_iota(jnp.int32, (_B, 4), 1)
    cf = (jnp.where(lane4 == 0, scale, 0.0)
          + jnp.where(lane4 == 1, density, 0.0)
          + jnp.where(lane4 == 2, adn, 0.0)
          + jnp.where(lane4 == 3, div, 0.0))
    cf_ref[...] = cf

    h = (_dot(pooled_g, W1_ref[0:_H, :], _DN_M)
         + _dot(pooled_a, W1_ref[_H:2 * _H, :], _DN_M)
         + _dot(cf, W1_ref[2 * _H:2 * _H + 4, :], _DN_M)
         + b1_ref[...])
    mu = jnp.mean(h, axis=1, keepdims=True)
    var = jnp.mean((h - mu) ** 2, axis=1, keepdims=True)
    h = (h - mu) * jax.lax.rsqrt(var + 1e-05) * g_ref[...] + be_ref[...]
    h = jnp.maximum(h, 0.0)
    h2 = jnp.maximum(_dot(h, W2_ref[...], _DN_M) + b2_ref[...], 0.0)
    raw_l = _dot(h2, W3l_ref[...], _DN_M) + b3l_ref[...]
    raw_g = _dot(h2, W3g_ref[...], _DN_M) + b3g_ref[...]
    m = jnp.maximum(raw_l, raw_g)
    el = jnp.exp(raw_l - m)
    eg = jnp.exp(raw_g - m)
    s = el + eg
    wl = el / s
    wg = eg / s
    wl_ref[...] = wl
    wg_ref[...] = wg

    wn_l = _dot(onehot, wl, _DN_M)
    wn_g = _dot(onehot, wg, _DN_M)
    out_ref[...] = wn_l * xg + wn_g * xa


def _sc_edge_hist(src, dst, batch):
    info = plsc.get_sparse_core_info()
    nc, ns, nl = info.num_cores, info.num_subcores, info.num_lanes
    nw = nc * ns
    epw = _E // nw
    mesh = plsc.VectorSubcoreMesh(core_axis_name="c", subcore_axis_name="s")

    @functools.partial(
        pl.kernel, mesh=mesh,
        compiler_params=pltpu.CompilerParams(needs_layout_passes=False),
        out_type=jax.ShapeDtypeStruct((nw, _B), _F32),
        scratch_types=[
            pltpu.VMEM((_N,), jnp.int32),
            pltpu.VMEM((epw,), jnp.int32),
            pltpu.VMEM((epw,), jnp.int32),
            pltpu.VMEM((_B,), _F32),
        ],
    )
    def sc_kern(src_hbm, dst_hbm, batch_hbm, out_hbm, batch_v, src_v, dst_v,
                cnt_v):
        wid = lax.axis_index("s") * nc + lax.axis_index("c")
        base = wid * epw
        pltpu.sync_copy(batch_hbm, batch_v)
        pltpu.sync_copy(src_hbm.at[pl.ds(base, epw)], src_v)
        pltpu.sync_copy(dst_hbm.at[pl.ds(base, epw)], dst_v)
        zeros = jnp.zeros((nl,), _F32)
        for k in range(_B // nl):
            cnt_v[pl.ds(k * nl, nl)] = zeros
        ones = jnp.ones((nl,), _F32)

        def body(i, carry):
            sl = src_v[pl.ds(i * nl, nl)]
            dl = dst_v[pl.ds(i * nl, nl)]
            bs = plsc.load_gather(batch_v, [sl])
            bd = plsc.load_gather(batch_v, [dl])
            plsc.addupdate_scatter(cnt_v, [bs], ones, mask=bs == bd)
            return carry

        lax.fori_loop(0, epw // nl, body, 0)
        pltpu.sync_copy(cnt_v, out_hbm.at[wid])

    return sc_kern(src, dst, batch)


def kernel(x_ggnn, x_appnp, edge_index, batch, W1, b1, gamma, beta,
           W2, b2, W3, b3):
    ne32 = _sc_edge_hist(edge_index[0], edge_index[1], batch)

    batch2d = batch.reshape(_N, 1)
    b1_2d = b1.reshape(1, _H)
    gamma_2d = gamma.reshape(1, _H)
    beta_2d = beta.reshape(1, _H)
    b2_2d = b2.reshape(1, _H // 2)
    W3l = W3[:, 0:1]
    W3g = W3[:, 1:2]
    b3l = b3[0:1].reshape(1, 1)
    b3g = b3[1:2].reshape(1, 1)

    out, wl, wg, cf = pl.pallas_call(
        _tc_body,
        out_shape=[
            jax.ShapeDtypeStruct((_N, _H), _F32),
            jax.ShapeDtypeStruct((_B, 1), _F32),
            jax.ShapeDtypeStruct((_B, 1), _F32),
            jax.ShapeDtypeStruct((_B, 4), _F32),
        ],
    )(ne32, batch2d, x_ggnn, x_appnp, W1, b1_2d, gamma_2d, beta_2d,
      W2, b2_2d, W3l, W3g, b3l, b3g)

    return out, wl.reshape(_B), wg.reshape(_B), cf


# speedup vs baseline: 74.1161x; 1.1124x over previous
import functools
import math

import jax
import jax.numpy as jnp
from jax import lax
from jax.experimental import pallas as pl
from jax.experimental.pallas import tpu as pltpu
from jax.experimental.pallas import tpu_sc as plsc

_N = 10000
_E = 320000
_B = 64
_H = 128
_MAX_NODES = 500
_F32 = jnp.float32

_DN_T = (((0,), (0,)), ((), ()))
_DN_M = (((1,), (0,)), ((), ()))
_PREC = jax.lax.Precision.DEFAULT


def _dot(a, b, dn):
    return jax.lax.dot_general(a, b, dn, precision=_PREC,
                               preferred_element_type=_F32)


def _tc_body(ne_ref, batch_ref, xg_ref, xa_ref, W1_ref, b1_ref, g_ref,
             be_ref, W2_ref, b2_ref, W3l_ref, W3g_ref, b3l_ref, b3g_ref,
             out_ref, wl_ref, wg_ref, cf_ref):
    xg = xg_ref[...]
    xa = xa_ref[...]
    bat = batch_ref[...]
    lanes = jax.lax.broadcasted_iota(jnp.int32, (1, _B), 1)
    onehot = (bat == lanes).astype(_F32)

    ones_n = jnp.full((_N, 1), 1.0, _F32)
    counts = _dot(onehot, ones_n, _DN_T)
    sum_g = _dot(onehot, xg, _DN_T)
    sum_a = _dot(onehot, xa, _DN_T)
    pooled_g = sum_g / counts
    pooled_a = sum_a / counts

    nw = ne_ref.shape[0]
    ne = _dot(ne_ref[...], jnp.full((nw, 1), 1.0, _F32), _DN_T)

    n_nodes = counts
    scale = jnp.log(n_nodes + 1.0) * (1.0 / math.log(_MAX_NODES + 1))
    density = ne / (n_nodes * (n_nodes - 1.0) + 1e-08)
    avg_degree = ne / (n_nodes + 1e-08)
    adn = jnp.minimum(avg_degree / 10.0, 1.0)

    num = jnp.sum(pooled_g * pooled_a, axis=1, keepdims=True)
    ng = jnp.sqrt(jnp.sum(pooled_g * pooled_g, axis=1, keepdims=True))
    na = jnp.sqrt(jnp.sum(pooled_a * pooled_a, axis=1, keepdims=True))
    cos = num / (jnp.maximum(ng, 1e-08) * jnp.maximum(na, 1e-08))
    div = (1.0 - cos) * 0.5

    lane4 = jax.lax.broadcasted_iota(jnp.int32, (_B, 4), 1)
    cf = (jnp.where(lane4 == 0, scale, 0.0)
          + jnp.where(lane4 == 1, density, 0.0)
          + jnp.where(lane4 == 2, adn, 0.0)
          + jnp.where(lane4 == 3, div, 0.0))
    cf_ref[...] = cf

    h = (_dot(pooled_g, W1_ref[0:_H, :], _DN_M)
         + _dot(pooled_a, W1_ref[_H:2 * _H, :], _DN_M)
         + _dot(cf, W1_ref[2 * _H:2 * _H + 4, :], _DN_M)
         + b1_ref[...])
    mu = jnp.mean(h, axis=1, keepdims=True)
    var = jnp.mean((h - mu) ** 2, axis=1, keepdims=True)
    h = (h - mu) * jax.lax.rsqrt(var + 1e-05) * g_ref[...] + be_ref[...]
    h = jnp.maximum(h, 0.0)
    h2 = jnp.maximum(_dot(h, W2_ref[...], _DN_M) + b2_ref[...], 0.0)
    raw_l = _dot(h2, W3l_ref[...], _DN_M) + b3l_ref[...]
    raw_g = _dot(h2, W3g_ref[...], _DN_M) + b3g_ref[...]
    m = jnp.maximum(raw_l, raw_g)
    el = jnp.exp(raw_l - m)
    eg = jnp.exp(raw_g - m)
    s = el + eg
    wl = el / s
    wg = eg / s
    wl_ref[...] = wl
    wg_ref[...] = wg

    wn_l = _dot(onehot, wl, _DN_M)
    wn_g = _dot(onehot, wg, _DN_M)
    out_ref[...] = wn_l * xg + wn_g * xa


def _sc_edge_hist(src, dst, batch):
    info = plsc.get_sparse_core_info()
    nc, ns, nl = info.num_cores, info.num_subcores, info.num_lanes
    nw = nc * ns
    epw = _E // nw
    mesh = plsc.VectorSubcoreMesh(core_axis_name="c", subcore_axis_name="s")

    @functools.partial(
        pl.kernel, mesh=mesh,
        compiler_params=pltpu.CompilerParams(needs_layout_passes=False),
        out_type=jax.ShapeDtypeStruct((nw, _B), _F32),
        scratch_types=[
            pltpu.VMEM((_N,), jnp.int32),
            pltpu.VMEM((epw,), jnp.int32),
            pltpu.VMEM((epw,), jnp.int32),
            pltpu.VMEM((_B,), _F32),
        ],
    )
    def sc_kern(src_hbm, dst_hbm, batch_hbm, out_hbm, batch_v, src_v, dst_v,
                cnt_v):
        wid = lax.axis_index("s") * nc + lax.axis_index("c")
        base = wid * epw
        pltpu.sync_copy(batch_hbm, batch_v)
        pltpu.sync_copy(src_hbm.at[pl.ds(base, epw)], src_v)
        pltpu.sync_copy(dst_hbm.at[pl.ds(base, epw)], dst_v)
        zeros = jnp.zeros((nl,), _F32)
        for k in range(_B // nl):
            cnt_v[pl.ds(k * nl, nl)] = zeros
        ones = jnp.ones((nl,), _F32)
        unroll = 5
        step = unroll * nl

        def body(i, carry):
            base_i = i * step
            gathered = []
            for u in range(unroll):
                sl = src_v[pl.ds(base_i + u * nl, nl)]
                dl = dst_v[pl.ds(base_i + u * nl, nl)]
                bs = plsc.load_gather(batch_v, [sl])
                bd = plsc.load_gather(batch_v, [dl])
                gathered.append((bs, bd))
            for bs, bd in gathered:
                plsc.addupdate_scatter(cnt_v, [bs], ones, mask=bs == bd)
            return carry

        lax.fori_loop(0, epw // step, body, 0)
        pltpu.sync_copy(cnt_v, out_hbm.at[wid])

    return sc_kern(src, dst, batch)


def kernel(x_ggnn, x_appnp, edge_index, batch, W1, b1, gamma, beta,
           W2, b2, W3, b3):
    ne32 = _sc_edge_hist(edge_index[0], edge_index[1], batch)

    batch2d = batch.reshape(_N, 1)
    b1_2d = b1.reshape(1, _H)
    gamma_2d = gamma.reshape(1, _H)
    beta_2d = beta.reshape(1, _H)
    b2_2d = b2.reshape(1, _H // 2)
    W3l = W3[:, 0:1]
    W3g = W3[:, 1:2]
    b3l = b3[0:1].reshape(1, 1)
    b3g = b3[1:2].reshape(1, 1)

    out, wl, wg, cf = pl.pallas_call(
        _tc_body,
        out_shape=[
            jax.ShapeDtypeStruct((_N, _H), _F32),
            jax.ShapeDtypeStruct((_B, 1), _F32),
            jax.ShapeDtypeStruct((_B, 1), _F32),
            jax.ShapeDtypeStruct((_B, 4), _F32),
        ],
    )(ne32, batch2d, x_ggnn, x_appnp, W1, b1_2d, gamma_2d, beta_2d,
      W2, b2_2d, W3l, W3g, b3l, b3g)

    return out, wl.reshape(_B), wg.reshape(_B), cf


# speedup vs baseline: 74.5442x; 1.0058x over previous
import functools
import math

import jax
import jax.numpy as jnp
from jax import lax
from jax.experimental import pallas as pl
from jax.experimental.pallas import tpu as pltpu
from jax.experimental.pallas import tpu_sc as plsc

_N = 10000
_E = 320000
_B = 64
_H = 128
_MAX_NODES = 500
_F32 = jnp.float32

_DN_T = (((0,), (0,)), ((), ()))
_DN_M = (((1,), (0,)), ((), ()))
_PREC = jax.lax.Precision.DEFAULT


def _dot(a, b, dn):
    return jax.lax.dot_general(a, b, dn, precision=_PREC,
                               preferred_element_type=_F32)


def _tc_body(ne_ref, batch_ref, xg_ref, xa_ref, W1_ref, b1_ref, g_ref,
             be_ref, W2_ref, b2_ref, W3l_ref, W3g_ref, b3l_ref, b3g_ref,
             out_ref, wl_ref, wg_ref, cf_ref):
    xg = xg_ref[...]
    xa = xa_ref[...]
    bat = batch_ref[...]
    lanes = jax.lax.broadcasted_iota(jnp.int32, (1, _B), 1)
    onehot = (bat == lanes).astype(_F32)

    ones_n = jnp.full((_N, 1), 1.0, _F32)
    counts = _dot(onehot, ones_n, _DN_T)
    sum_g = _dot(onehot, xg, _DN_T)
    sum_a = _dot(onehot, xa, _DN_T)
    pooled_g = sum_g / counts
    pooled_a = sum_a / counts

    nw = ne_ref.shape[0]
    ne = _dot(ne_ref[...], jnp.full((nw, 1), 1.0, _F32), _DN_T)

    n_nodes = counts
    scale = jnp.log(n_nodes + 1.0) * (1.0 / math.log(_MAX_NODES + 1))
    density = ne / (n_nodes * (n_nodes - 1.0) + 1e-08)
    avg_degree = ne / (n_nodes + 1e-08)
    adn = jnp.minimum(avg_degree / 10.0, 1.0)

    num = jnp.sum(pooled_g * pooled_a, axis=1, keepdims=True)
    ng = jnp.sqrt(jnp.sum(pooled_g * pooled_g, axis=1, keepdims=True))
    na = jnp.sqrt(jnp.sum(pooled_a * pooled_a, axis=1, keepdims=True))
    cos = num / (jnp.maximum(ng, 1e-08) * jnp.maximum(na, 1e-08))
    div = (1.0 - cos) * 0.5

    lane4 = jax.lax.broadcasted_iota(jnp.int32, (_B, 4), 1)
    cf = (jnp.where(lane4 == 0, scale, 0.0)
          + jnp.where(lane4 == 1, density, 0.0)
          + jnp.where(lane4 == 2, adn, 0.0)
          + jnp.where(lane4 == 3, div, 0.0))
    cf_ref[...] = cf

    h = (_dot(pooled_g, W1_ref[0:_H, :], _DN_M)
         + _dot(pooled_a, W1_ref[_H:2 * _H, :], _DN_M)
         + _dot(cf, W1_ref[2 * _H:2 * _H + 4, :], _DN_M)
         + b1_ref[...])
    mu = jnp.mean(h, axis=1, keepdims=True)
    var = jnp.mean((h - mu) ** 2, axis=1, keepdims=True)
    h = (h - mu) * jax.lax.rsqrt(var + 1e-05) * g_ref[...] + be_ref[...]
    h = jnp.maximum(h, 0.0)
    h2 = jnp.maximum(_dot(h, W2_ref[...], _DN_M) + b2_ref[...], 0.0)
    raw_l = _dot(h2, W3l_ref[...], _DN_M) + b3l_ref[...]
    raw_g = _dot(h2, W3g_ref[...], _DN_M) + b3g_ref[...]
    m = jnp.maximum(raw_l, raw_g)
    el = jnp.exp(raw_l - m)
    eg = jnp.exp(raw_g - m)
    s = el + eg
    wl = el / s
    wg = eg / s
    wl_ref[...] = wl
    wg_ref[...] = wg

    wn_l = _dot(onehot, wl, _DN_M)
    wn_g = _dot(onehot, wg, _DN_M)
    out_ref[...] = wn_l * xg + wn_g * xa


def _sc_edge_hist(src, dst, batch):
    info = plsc.get_sparse_core_info()
    nc, ns, nl = info.num_cores, info.num_subcores, info.num_lanes
    nw = nc * ns
    epw = _E // nw
    mesh = plsc.VectorSubcoreMesh(core_axis_name="c", subcore_axis_name="s")

    @functools.partial(
        pl.kernel, mesh=mesh,
        compiler_params=pltpu.CompilerParams(needs_layout_passes=False),
        out_type=jax.ShapeDtypeStruct((nw, _B), _F32),
        scratch_types=[
            pltpu.VMEM((_N,), jnp.int32),
            pltpu.VMEM((epw,), jnp.int32),
            pltpu.VMEM((epw,), jnp.int32),
            pltpu.VMEM((_B,), _F32),
            pltpu.VMEM((_B,), _F32),
        ],
    )
    def sc_kern(src_hbm, dst_hbm, batch_hbm, out_hbm, batch_v, src_v, dst_v,
                cnt_v, cnt2_v):
        wid = lax.axis_index("s") * nc + lax.axis_index("c")
        base = wid * epw
        pltpu.sync_copy(batch_hbm, batch_v)
        pltpu.sync_copy(src_hbm.at[pl.ds(base, epw)], src_v)
        pltpu.sync_copy(dst_hbm.at[pl.ds(base, epw)], dst_v)
        zeros = jnp.zeros((nl,), _F32)
        for k in range(_B // nl):
            cnt_v[pl.ds(k * nl, nl)] = zeros
            cnt2_v[pl.ds(k * nl, nl)] = zeros
        ones = jnp.ones((nl,), _F32)
        unroll = 10
        step = unroll * nl

        def body(i, carry):
            base_i = i * step
            gathered = []
            for u in range(unroll):
                sl = src_v[pl.ds(base_i + u * nl, nl)]
                dl = dst_v[pl.ds(base_i + u * nl, nl)]
                bs = plsc.load_gather(batch_v, [sl])
                bd = plsc.load_gather(batch_v, [dl])
                gathered.append((bs, bd))
            for u, (bs, bd) in enumerate(gathered):
                tgt = cnt_v if u % 2 == 0 else cnt2_v
                plsc.addupdate_scatter(tgt, [bs], ones, mask=bs == bd)
            return carry

        lax.fori_loop(0, epw // step, body, 0)
        for k in range(_B // nl):
            sl = pl.ds(k * nl, nl)
            cnt_v[sl] = cnt_v[sl] + cnt2_v[sl]
        pltpu.sync_copy(cnt_v, out_hbm.at[wid])

    return sc_kern(src, dst, batch)


def kernel(x_ggnn, x_appnp, edge_index, batch, W1, b1, gamma, beta,
           W2, b2, W3, b3):
    ne32 = _sc_edge_hist(edge_index[0], edge_index[1], batch)

    batch2d = batch.reshape(_N, 1)
    b1_2d = b1.reshape(1, _H)
    gamma_2d = gamma.reshape(1, _H)
    beta_2d = beta.reshape(1, _H)
    b2_2d = b2.reshape(1, _H // 2)
    W3l = W3[:, 0:1]
    W3g = W3[:, 1:2]
    b3l = b3[0:1].reshape(1, 1)
    b3g = b3[1:2].reshape(1, 1)

    out, wl, wg, cf = pl.pallas_call(
        _tc_body,
        out_shape=[
            jax.ShapeDtypeStruct((_N, _H), _F32),
            jax.ShapeDtypeStruct((_B, 1), _F32),
            jax.ShapeDtypeStruct((_B, 1), _F32),
            jax.ShapeDtypeStruct((_B, 4), _F32),
        ],
    )(ne32, batch2d, x_ggnn, x_appnp, W1, b1_2d, gamma_2d, beta_2d,
      W2, b2_2d, W3l, W3g, b3l, b3g)

    return out, wl.reshape(_B), wg.reshape(_B), cf


# speedup vs baseline: 85.2707x; 1.1439x over previous
import functools
import math

import jax
import jax.numpy as jnp
from jax import lax
from jax.experimental import pallas as pl
from jax.experimental.pallas import tpu as pltpu
from jax.experimental.pallas import tpu_sc as plsc

_N = 10000
_E = 320000
_B = 64
_H = 128
_MAX_NODES = 500
_F32 = jnp.float32

_BLK = 1000
_NB = _N // _BLK

_DN_T = (((0,), (0,)), ((), ()))
_DN_M = (((1,), (0,)), ((), ()))


def _dot(a, b, dn):
    return jax.lax.dot_general(a, b, dn, preferred_element_type=_F32)


def _tc_body(ne_ref, batch_ref, xg_ref, xa_ref, W1_ref, b1_ref, g_ref,
             be_ref, W2_ref, b2_ref, W3l_ref, W3g_ref, b3l_ref, b3g_ref,
             out_ref, wl_ref, wg_ref, cf_ref,
             accg, acca, acccnt, wls, wgs):
    p = pl.program_id(0)
    i = pl.program_id(1)
    bat = batch_ref[...].reshape(1, _BLK)
    rows = jax.lax.broadcasted_iota(jnp.int32, (_B, 1), 0)
    oh = (bat == rows).astype(_F32)

    @pl.when(p == 0)
    def _phase0():
        @pl.when(i == 0)
        def _init():
            accg[...] = jnp.zeros_like(accg)
            acca[...] = jnp.zeros_like(acca)
            acccnt[...] = jnp.zeros_like(acccnt)

        accg[...] += _dot(oh, xg_ref[...], _DN_M)
        acca[...] += _dot(oh, xa_ref[...], _DN_M)
        acccnt[...] += _dot(oh, jnp.full((_BLK, 1), 1.0, _F32), _DN_M)

        @pl.when(i == _NB - 1)
        def _mlp():
            counts = acccnt[...]
            pooled_g = accg[...] / counts
            pooled_a = acca[...] / counts
            nw = ne_ref.shape[0]
            ne = _dot(ne_ref[...], jnp.full((nw, 1), 1.0, _F32), _DN_T)

            n_nodes = counts
            scale = jnp.log(n_nodes + 1.0) * (1.0 / math.log(_MAX_NODES + 1))
            density = ne / (n_nodes * (n_nodes - 1.0) + 1e-08)
            avg_degree = ne / (n_nodes + 1e-08)
            adn = jnp.minimum(avg_degree / 10.0, 1.0)

            num = jnp.sum(pooled_g * pooled_a, axis=1, keepdims=True)
            ngn = jnp.sqrt(jnp.sum(pooled_g * pooled_g, axis=1, keepdims=True))
            nan_ = jnp.sqrt(jnp.sum(pooled_a * pooled_a, axis=1, keepdims=True))
            cos = num / (jnp.maximum(ngn, 1e-08) * jnp.maximum(nan_, 1e-08))
            div = (1.0 - cos) * 0.5

            lane4 = jax.lax.broadcasted_iota(jnp.int32, (_B, 4), 1)
            cf = (jnp.where(lane4 == 0, scale, 0.0)
                  + jnp.where(lane4 == 1, density, 0.0)
                  + jnp.where(lane4 == 2, adn, 0.0)
                  + jnp.where(lane4 == 3, div, 0.0))
            cf_ref[...] = cf

            h = (_dot(pooled_g, W1_ref[0:_H, :], _DN_M)
                 + _dot(pooled_a, W1_ref[_H:2 * _H, :], _DN_M)
                 + _dot(cf, W1_ref[2 * _H:2 * _H + 4, :], _DN_M)
                 + b1_ref[...])
            mu = jnp.mean(h, axis=1, keepdims=True)
            var = jnp.mean((h - mu) ** 2, axis=1, keepdims=True)
            h = (h - mu) * jax.lax.rsqrt(var + 1e-05) * g_ref[...] + be_ref[...]
            h = jnp.maximum(h, 0.0)
            h2 = jnp.maximum(_dot(h, W2_ref[...], _DN_M) + b2_ref[...], 0.0)
            raw_l = _dot(h2, W3l_ref[...], _DN_M) + b3l_ref[...]
            raw_g = _dot(h2, W3g_ref[...], _DN_M) + b3g_ref[...]
            m = jnp.maximum(raw_l, raw_g)
            el = jnp.exp(raw_l - m)
            eg = jnp.exp(raw_g - m)
            s = el + eg
            wl = el / s
            wg = eg / s
            wl_ref[...] = wl
            wg_ref[...] = wg
            wls[...] = wl
            wgs[...] = wg

    @pl.when(p == 1)
    def _phase1():
        wn_l = _dot(oh, wls[...], _DN_T)
        wn_g = _dot(oh, wgs[...], _DN_T)
        out_ref[...] = wn_l * xg_ref[...] + wn_g * xa_ref[...]


def _sc_edge_hist(edge_index, batch):
    info = plsc.get_sparse_core_info()
    nc, ns, nl = info.num_cores, info.num_subcores, info.num_lanes
    nw = nc * ns
    tiles = _E // 128
    tpw = tiles // nw
    cols = tpw * 128
    ntail = tiles - tpw * nw
    tail0 = tpw * nw * 128
    mesh = plsc.VectorSubcoreMesh(core_axis_name="c", subcore_axis_name="s")

    @functools.partial(
        pl.kernel, mesh=mesh,
        compiler_params=pltpu.CompilerParams(needs_layout_passes=False),
        out_type=jax.ShapeDtypeStruct((nw, _B), _F32),
        scratch_types=[
            pltpu.VMEM((_N,), jnp.int32),
            pltpu.VMEM((2, cols), jnp.int32),
            pltpu.VMEM((2, 128), jnp.int32),
            pltpu.VMEM((_B,), _F32),
            pltpu.VMEM((_B,), _F32),
        ],
    )
    def sc_kern(edge_hbm, batch_hbm, out_hbm, batch_v, ev_v, tail_v,
                cnt_v, cnt2_v):
        wid = lax.axis_index("s") * nc + lax.axis_index("c")
        pltpu.sync_copy(batch_hbm, batch_v)
        pltpu.sync_copy(edge_hbm.at[:, pl.ds(wid * cols, cols)], ev_v)
        zeros = jnp.zeros((nl,), _F32)
        for k in range(_B // nl):
            cnt_v[pl.ds(k * nl, nl)] = zeros
            cnt2_v[pl.ds(k * nl, nl)] = zeros
        ones = jnp.ones((nl,), _F32)
        unroll = 8
        step = unroll * nl

        def make_body(ev_ref):
            def body(j, carry):
                base_j = j * step
                gathered = []
                for u in range(unroll):
                    sl = ev_ref[0, pl.ds(base_j + u * nl, nl)]
                    dl = ev_ref[1, pl.ds(base_j + u * nl, nl)]
                    bs = plsc.load_gather(batch_v, [sl])
                    bd = plsc.load_gather(batch_v, [dl])
                    gathered.append((bs, bd))
                for u, (bs, bd) in enumerate(gathered):
                    tgt = cnt_v if u % 2 == 0 else cnt2_v
                    plsc.addupdate_scatter(tgt, [bs], ones, mask=bs == bd)
                return carry
            return body

        lax.fori_loop(0, cols // step, make_body(ev_v), 0)

        @pl.when(wid < ntail)
        def _tail():
            pltpu.sync_copy(edge_hbm.at[:, pl.ds(tail0 + wid * 128, 128)],
                            tail_v)
            lax.fori_loop(0, 1, make_body(tail_v), 0)

        for k in range(_B // nl):
            sl = pl.ds(k * nl, nl)
            cnt_v[sl] = cnt_v[sl] + cnt2_v[sl]
        pltpu.sync_copy(cnt_v, out_hbm.at[wid])

    return sc_kern(edge_index, batch)


def kernel(x_ggnn, x_appnp, edge_index, batch, W1, b1, gamma, beta,
           W2, b2, W3, b3):
    ne32 = _sc_edge_hist(edge_index, batch)

    b1_2d = b1.reshape(1, _H)
    gamma_2d = gamma.reshape(1, _H)
    beta_2d = beta.reshape(1, _H)
    b2_2d = b2.reshape(1, _H // 2)
    W3l = W3[:, 0:1]
    W3g = W3[:, 1:2]
    b3l = b3[0:1].reshape(1, 1)
    b3g = b3[1:2].reshape(1, 1)

    nw = ne32.shape[0]
    full = lambda p, i: (0, 0)
    blk_x = pl.BlockSpec((_BLK, _H), lambda p, i: (i, 0))
    out_map = pl.BlockSpec((_BLK, _H), lambda p, i: (jnp.where(p == 1, i, 0), 0))

    out, wl, wg, cf = pl.pallas_call(
        _tc_body,
        grid=(2, _NB),
        in_specs=[
            pl.BlockSpec((nw, _B), full),
            pl.BlockSpec((1, 1, _BLK), lambda p, i: (i, 0, 0)),
            blk_x,
            blk_x,
            pl.BlockSpec((2 * _H + 4, _H), full),
            pl.BlockSpec((1, _H), full),
            pl.BlockSpec((1, _H), full),
            pl.BlockSpec((1, _H), full),
            pl.BlockSpec((_H, _H // 2), full),
            pl.BlockSpec((1, _H // 2), full),
            pl.BlockSpec((_B, 1), full),
            pl.BlockSpec((_B, 1), full),
            pl.BlockSpec((1, 1), full),
            pl.BlockSpec((1, 1), full),
        ],
        out_specs=[
            out_map,
            pl.BlockSpec((_B, 1), full),
            pl.BlockSpec((_B, 1), full),
            pl.BlockSpec((_B, 4), full),
        ],
        out_shape=[
            jax.ShapeDtypeStruct((_N, _H), _F32),
            jax.ShapeDtypeStruct((_B, 1), _F32),
            jax.ShapeDtypeStruct((_B, 1), _F32),
            jax.ShapeDtypeStruct((_B, 4), _F32),
        ],
        scratch_shapes=[
            pltpu.VMEM((_B, _H), _F32),
            pltpu.VMEM((_B, _H), _F32),
            pltpu.VMEM((_B, 1), _F32),
            pltpu.VMEM((_B, 1), _F32),
            pltpu.VMEM((_B, 1), _F32),
        ],
    )(ne32, batch.reshape(_NB, 1, _BLK), x_ggnn, x_appnp, W1, b1_2d,
      gamma_2d, beta_2d, W2, b2_2d, W3l, W3g, b3l, b3g)

    return out, wl.reshape(_B), wg.reshape(_B), cf


# speedup vs baseline: 90.5971x; 1.0625x over previous
import functools
import math

import jax
import jax.numpy as jnp
from jax import lax
from jax.experimental import pallas as pl
from jax.experimental.pallas import tpu as pltpu
from jax.experimental.pallas import tpu_sc as plsc

_N = 10000
_E = 320000
_B = 64
_H = 128
_MAX_NODES = 500
_F32 = jnp.float32

_BLK = 1000
_NB = _N // _BLK

_DN_T = (((0,), (0,)), ((), ()))
_DN_M = (((1,), (0,)), ((), ()))


def _dot(a, b, dn):
    return jax.lax.dot_general(a, b, dn, preferred_element_type=_F32)


def _tc_body(ne_ref, batch_ref, xg_ref, xa_ref, W1_ref, b1_ref, g_ref,
             be_ref, W2_ref, b2_ref, W3l_ref, W3g_ref, b3l_ref, b3g_ref,
             out_ref, wl_ref, wg_ref, cf_ref,
             accg, acca, acccnt, wls, wgs, xg_all, xa_all):
    p = pl.program_id(0)
    i = pl.program_id(1)
    bat = batch_ref[...].reshape(1, _BLK)
    rows = jax.lax.broadcasted_iota(jnp.int32, (_B, 1), 0)
    same = bat == rows

    @pl.when(p == 0)
    def _phase0():
        @pl.when(i == 0)
        def _init():
            accg[...] = jnp.zeros_like(accg)
            acca[...] = jnp.zeros_like(acca)
            acccnt[...] = jnp.zeros_like(acccnt)

        ohb = same.astype(jnp.bfloat16)
        xgb = xg_ref[...]
        xab = xa_ref[...]
        xg_all[pl.ds(i * _BLK, _BLK), :] = xgb
        xa_all[pl.ds(i * _BLK, _BLK), :] = xab
        accg[...] += _dot(ohb, xgb.astype(jnp.bfloat16), _DN_M)
        acca[...] += _dot(ohb, xab.astype(jnp.bfloat16), _DN_M)
        acccnt[...] += _dot(ohb, jnp.full((_BLK, 1), 1.0, jnp.bfloat16),
                            _DN_M)

        @pl.when(i == _NB - 1)
        def _mlp():
            counts = acccnt[...]
            pooled_g = accg[...] / counts
            pooled_a = acca[...] / counts
            nw = ne_ref.shape[0]
            ne = _dot(ne_ref[...], jnp.full((nw, 1), 1.0, _F32), _DN_T)

            n_nodes = counts
            scale = jnp.log(n_nodes + 1.0) * (1.0 / math.log(_MAX_NODES + 1))
            density = ne / (n_nodes * (n_nodes - 1.0) + 1e-08)
            avg_degree = ne / (n_nodes + 1e-08)
            adn = jnp.minimum(avg_degree / 10.0, 1.0)

            num = jnp.sum(pooled_g * pooled_a, axis=1, keepdims=True)
            ngn = jnp.sqrt(jnp.sum(pooled_g * pooled_g, axis=1, keepdims=True))
            nan_ = jnp.sqrt(jnp.sum(pooled_a * pooled_a, axis=1, keepdims=True))
            cos = num / (jnp.maximum(ngn, 1e-08) * jnp.maximum(nan_, 1e-08))
            div = (1.0 - cos) * 0.5

            lane4 = jax.lax.broadcasted_iota(jnp.int32, (_B, 4), 1)
            cf = (jnp.where(lane4 == 0, scale, 0.0)
                  + jnp.where(lane4 == 1, density, 0.0)
                  + jnp.where(lane4 == 2, adn, 0.0)
                  + jnp.where(lane4 == 3, div, 0.0))
            cf_ref[...] = cf

            h = (_dot(pooled_g, W1_ref[0:_H, :], _DN_M)
                 + _dot(pooled_a, W1_ref[_H:2 * _H, :], _DN_M)
                 + _dot(cf, W1_ref[2 * _H:2 * _H + 4, :], _DN_M)
                 + b1_ref[...])
            mu = jnp.mean(h, axis=1, keepdims=True)
            var = jnp.mean((h - mu) ** 2, axis=1, keepdims=True)
            h = (h - mu) * jax.lax.rsqrt(var + 1e-05) * g_ref[...] + be_ref[...]
            h = jnp.maximum(h, 0.0)
            h2 = jnp.maximum(_dot(h, W2_ref[...], _DN_M) + b2_ref[...], 0.0)
            raw_l = _dot(h2, W3l_ref[...], _DN_M) + b3l_ref[...]
            raw_g = _dot(h2, W3g_ref[...], _DN_M) + b3g_ref[...]
            m = jnp.maximum(raw_l, raw_g)
            el = jnp.exp(raw_l - m)
            eg = jnp.exp(raw_g - m)
            s = el + eg
            wl = el / s
            wg = eg / s
            wl_ref[...] = wl
            wg_ref[...] = wg
            wls[...] = wl
            wgs[...] = wg

    @pl.when(p == 1)
    def _phase1():
        ohf = same.astype(_F32)
        wn_l = _dot(ohf, wls[...], _DN_T)
        wn_g = _dot(ohf, wgs[...], _DN_T)
        sl = pl.ds(i * _BLK, _BLK)
        out_ref[...] = wn_l * xg_all[sl, :] + wn_g * xa_all[sl, :]


def _sc_edge_hist(edge_index, batch):
    info = plsc.get_sparse_core_info()
    nc, ns, nl = info.num_cores, info.num_subcores, info.num_lanes
    nw = nc * ns
    tiles = _E // 128
    tpw = tiles // nw
    cols = tpw * 128
    ntail = tiles - tpw * nw
    tail0 = tpw * nw * 128
    mesh = plsc.VectorSubcoreMesh(core_axis_name="c", subcore_axis_name="s")

    @functools.partial(
        pl.kernel, mesh=mesh,
        compiler_params=pltpu.CompilerParams(needs_layout_passes=False),
        out_type=jax.ShapeDtypeStruct((nw, _B), _F32),
        scratch_types=[
            pltpu.VMEM((_N,), jnp.int32),
            pltpu.VMEM((2, cols), jnp.int32),
            pltpu.VMEM((2, 128), jnp.int32),
            pltpu.VMEM((_B,), _F32),
            pltpu.VMEM((_B,), _F32),
        ],
    )
    def sc_kern(edge_hbm, batch_hbm, out_hbm, batch_v, ev_v, tail_v,
                cnt_v, cnt2_v):
        wid = lax.axis_index("s") * nc + lax.axis_index("c")
        pltpu.sync_copy(batch_hbm, batch_v)
        pltpu.sync_copy(edge_hbm.at[:, pl.ds(wid * cols, cols)], ev_v)
        zeros = jnp.zeros((nl,), _F32)
        for k in range(_B // nl):
            cnt_v[pl.ds(k * nl, nl)] = zeros
            cnt2_v[pl.ds(k * nl, nl)] = zeros
        ones = jnp.ones((nl,), _F32)
        unroll = 8
        step = unroll * nl

        def make_body(ev_ref):
            def body(j, carry):
                base_j = j * step
                gathered = []
                for u in range(unroll):
                    sl = ev_ref[0, pl.ds(base_j + u * nl, nl)]
                    dl = ev_ref[1, pl.ds(base_j + u * nl, nl)]
                    bs = plsc.load_gather(batch_v, [sl])
                    bd = plsc.load_gather(batch_v, [dl])
                    gathered.append((bs, bd))
                for u, (bs, bd) in enumerate(gathered):
                    tgt = cnt_v if u % 2 == 0 else cnt2_v
                    plsc.addupdate_scatter(tgt, [bs], ones, mask=bs == bd)
                return carry
            return body

        lax.fori_loop(0, cols // step, make_body(ev_v), 0)

        @pl.when(wid < ntail)
        def _tail():
            pltpu.sync_copy(edge_hbm.at[:, pl.ds(tail0 + wid * 128, 128)],
                            tail_v)
            lax.fori_loop(0, 1, make_body(tail_v), 0)

        for k in range(_B // nl):
            sl = pl.ds(k * nl, nl)
            cnt_v[sl] = cnt_v[sl] + cnt2_v[sl]
        pltpu.sync_copy(cnt_v, out_hbm.at[wid])

    return sc_kern(edge_index, batch)


def kernel(x_ggnn, x_appnp, edge_index, batch, W1, b1, gamma, beta,
           W2, b2, W3, b3):
    ne32 = _sc_edge_hist(edge_index, batch)

    b1_2d = b1.reshape(1, _H)
    gamma_2d = gamma.reshape(1, _H)
    beta_2d = beta.reshape(1, _H)
    b2_2d = b2.reshape(1, _H // 2)
    W3l = W3[:, 0:1]
    W3g = W3[:, 1:2]
    b3l = b3[0:1].reshape(1, 1)
    b3g = b3[1:2].reshape(1, 1)

    nw = ne32.shape[0]
    full = lambda p, i: (0, 0)
    blk_x = pl.BlockSpec((_BLK, _H),
                         lambda p, i: (jnp.where(p == 0, i, _NB - 1), 0))
    out_map = pl.BlockSpec((_BLK, _H), lambda p, i: (jnp.where(p == 1, i, 0), 0))

    out, wl, wg, cf = pl.pallas_call(
        _tc_body,
        grid=(2, _NB),
        in_specs=[
            pl.BlockSpec((nw, _B), full),
            pl.BlockSpec((1, 1, _BLK), lambda p, i: (i, 0, 0)),
            blk_x,
            blk_x,
            pl.BlockSpec((2 * _H + 4, _H), full),
            pl.BlockSpec((1, _H), full),
            pl.BlockSpec((1, _H), full),
            pl.BlockSpec((1, _H), full),
            pl.BlockSpec((_H, _H // 2), full),
            pl.BlockSpec((1, _H // 2), full),
            pl.BlockSpec((_B, 1), full),
            pl.BlockSpec((_B, 1), full),
            pl.BlockSpec((1, 1), full),
            pl.BlockSpec((1, 1), full),
        ],
        out_specs=[
            out_map,
            pl.BlockSpec((_B, 1), full),
            pl.BlockSpec((_B, 1), full),
            pl.BlockSpec((_B, 4), full),
        ],
        out_shape=[
            jax.ShapeDtypeStruct((_N, _H), _F32),
            jax.ShapeDtypeStruct((_B, 1), _F32),
            jax.ShapeDtypeStruct((_B, 1), _F32),
            jax.ShapeDtypeStruct((_B, 4), _F32),
        ],
        scratch_shapes=[
            pltpu.VMEM((_B, _H), _F32),
            pltpu.VMEM((_B, _H), _F32),
            pltpu.VMEM((_B, 1), _F32),
            pltpu.VMEM((_B, 1), _F32),
            pltpu.VMEM((_B, 1), _F32),
            pltpu.VMEM((_N, _H), _F32),
            pltpu.VMEM((_N, _H), _F32),
        ],
    )(ne32, batch.reshape(_NB, 1, _BLK), x_ggnn, x_appnp, W1, b1_2d,
      gamma_2d, beta_2d, W2, b2_2d, W3l, W3g, b3l, b3g)

    return out, wl.reshape(_B), wg.reshape(_B), cf


# speedup vs baseline: 94.0907x; 1.0386x over previous
import functools
import math

import jax
import jax.numpy as jnp
from jax import lax
from jax.experimental import pallas as pl
from jax.experimental.pallas import tpu as pltpu
from jax.experimental.pallas import tpu_sc as plsc

_N = 10000
_E = 320000
_B = 64
_H = 128
_MAX_NODES = 500
_F32 = jnp.float32

_BLK = 1000
_NB = _N // _BLK

_DN_T = (((0,), (0,)), ((), ()))
_DN_M = (((1,), (0,)), ((), ()))


def _dot(a, b, dn):
    return jax.lax.dot_general(a, b, dn, preferred_element_type=_F32)


def _oh_mask(batch_ref):
    bat = batch_ref[...].reshape(1, _BLK)
    rows = jax.lax.broadcasted_iota(jnp.int32, (_B, 1), 0)
    return bat == rows


def _sums_body(batch_ref, xg_ref, xa_ref, sums_ref, cnt_ref, accx, acccnt):
    i = pl.program_id(0)
    same = _oh_mask(batch_ref)

    @pl.when(i == 0)
    def _init():
        accx[...] = jnp.zeros_like(accx)
        acccnt[...] = jnp.zeros_like(acccnt)

    ohb = same.astype(jnp.bfloat16)
    xcat = jnp.concatenate([xg_ref[...].astype(jnp.bfloat16),
                            xa_ref[...].astype(jnp.bfloat16)], axis=1)
    accx[...] += _dot(ohb, xcat, _DN_M)
    acccnt[...] += jnp.sum(same.astype(_F32), axis=1, keepdims=True)

    @pl.when(i == _NB - 1)
    def _emit():
        sums_ref[...] = accx[...]
        cnt_ref[...] = acccnt[...]


def _gate_fuse_body(ne_ref, sums_ref, cnt_ref, batch_ref, xg_ref, xa_ref,
                    W1_ref, b1_ref, g_ref, be_ref, W2_ref, b2_ref,
                    W3l_ref, W3g_ref, b3l_ref, b3g_ref,
                    out_ref, wl_ref, wg_ref, cf_ref, wls, wgs):
    i = pl.program_id(0)

    @pl.when(i == 0)
    def _mlp():
        counts = cnt_ref[...]
        pooled_g = sums_ref[:, 0:_H] / counts
        pooled_a = sums_ref[:, _H:2 * _H] / counts
        nw = ne_ref.shape[0]
        ne = _dot(ne_ref[...], jnp.full((nw, 1), 1.0, _F32), _DN_T)

        n_nodes = counts
        scale = jnp.log(n_nodes + 1.0) * (1.0 / math.log(_MAX_NODES + 1))
        density = ne / (n_nodes * (n_nodes - 1.0) + 1e-08)
        avg_degree = ne / (n_nodes + 1e-08)
        adn = jnp.minimum(avg_degree / 10.0, 1.0)

        num = jnp.sum(pooled_g * pooled_a, axis=1, keepdims=True)
        ngn = jnp.sqrt(jnp.sum(pooled_g * pooled_g, axis=1, keepdims=True))
        nan_ = jnp.sqrt(jnp.sum(pooled_a * pooled_a, axis=1, keepdims=True))
        cos = num / (jnp.maximum(ngn, 1e-08) * jnp.maximum(nan_, 1e-08))
        div = (1.0 - cos) * 0.5

        lane4 = jax.lax.broadcasted_iota(jnp.int32, (_B, 4), 1)
        cf = (jnp.where(lane4 == 0, scale, 0.0)
              + jnp.where(lane4 == 1, density, 0.0)
              + jnp.where(lane4 == 2, adn, 0.0)
              + jnp.where(lane4 == 3, div, 0.0))
        cf_ref[...] = cf

        h = (_dot(pooled_g, W1_ref[0:_H, :], _DN_M)
             + _dot(pooled_a, W1_ref[_H:2 * _H, :], _DN_M)
             + _dot(cf, W1_ref[2 * _H:2 * _H + 4, :], _DN_M)
             + b1_ref[...])
        mu = jnp.mean(h, axis=1, keepdims=True)
        var = jnp.mean((h - mu) ** 2, axis=1, keepdims=True)
        h = (h - mu) * jax.lax.rsqrt(var + 1e-05) * g_ref[...] + be_ref[...]
        h = jnp.maximum(h, 0.0)
        h2 = jnp.maximum(_dot(h, W2_ref[...], _DN_M) + b2_ref[...], 0.0)
        raw_l = _dot(h2, W3l_ref[...], _DN_M) + b3l_ref[...]
        raw_g = _dot(h2, W3g_ref[...], _DN_M) + b3g_ref[...]
        m = jnp.maximum(raw_l, raw_g)
        el = jnp.exp(raw_l - m)
        eg = jnp.exp(raw_g - m)
        s = el + eg
        wl = el / s
        wg = eg / s
        wl_ref[...] = wl
        wg_ref[...] = wg
        wls[...] = wl
        wgs[...] = wg

    ohf = _oh_mask(batch_ref).astype(_F32)
    wn_l = _dot(ohf, wls[...], _DN_T)
    wn_g = _dot(ohf, wgs[...], _DN_T)
    out_ref[...] = wn_l * xg_ref[...] + wn_g * xa_ref[...]


def _sc_edge_hist(edge_index, batch):
    info = plsc.get_sparse_core_info()
    nc, ns, nl = info.num_cores, info.num_subcores, info.num_lanes
    nw = nc * ns
    tiles = _E // 128
    tpw = tiles // nw
    cols = tpw * 128
    ntail = tiles - tpw * nw
    tail0 = tpw * nw * 128
    mesh = plsc.VectorSubcoreMesh(core_axis_name="c", subcore_axis_name="s")

    @functools.partial(
        pl.kernel, mesh=mesh,
        compiler_params=pltpu.CompilerParams(needs_layout_passes=False),
        out_type=jax.ShapeDtypeStruct((nw, _B), _F32),
        scratch_types=[
            pltpu.VMEM((_N,), jnp.int32),
            pltpu.VMEM((2, cols), jnp.int32),
            pltpu.VMEM((2, 128), jnp.int32),
            pltpu.VMEM((_B,), _F32),
            pltpu.VMEM((_B,), _F32),
        ],
    )
    def sc_kern(edge_hbm, batch_hbm, out_hbm, batch_v, ev_v, tail_v,
                cnt_v, cnt2_v):
        wid = lax.axis_index("s") * nc + lax.axis_index("c")
        pltpu.sync_copy(batch_hbm, batch_v)
        pltpu.sync_copy(edge_hbm.at[:, pl.ds(wid * cols, cols)], ev_v)
        zeros = jnp.zeros((nl,), _F32)
        for k in range(_B // nl):
            cnt_v[pl.ds(k * nl, nl)] = zeros
            cnt2_v[pl.ds(k * nl, nl)] = zeros
        ones = jnp.ones((nl,), _F32)
        unroll = 8
        step = unroll * nl

        def make_body(ev_ref):
            def body(j, carry):
                base_j = j * step
                gathered = []
                for u in range(unroll):
                    sl = ev_ref[0, pl.ds(base_j + u * nl, nl)]
                    dl = ev_ref[1, pl.ds(base_j + u * nl, nl)]
                    bs = plsc.load_gather(batch_v, [sl])
                    bd = plsc.load_gather(batch_v, [dl])
                    gathered.append((bs, bd))
                for u, (bs, bd) in enumerate(gathered):
                    tgt = cnt_v if u % 2 == 0 else cnt2_v
                    plsc.addupdate_scatter(tgt, [bs], ones, mask=bs == bd)
                return carry
            return body

        lax.fori_loop(0, cols // step, make_body(ev_v), 0)

        @pl.when(wid < ntail)
        def _tail():
            pltpu.sync_copy(edge_hbm.at[:, pl.ds(tail0 + wid * 128, 128)],
                            tail_v)
            lax.fori_loop(0, 1, make_body(tail_v), 0)

        for k in range(_B // nl):
            sl = pl.ds(k * nl, nl)
            cnt_v[sl] = cnt_v[sl] + cnt2_v[sl]
        pltpu.sync_copy(cnt_v, out_hbm.at[wid])

    return sc_kern(edge_index, batch)


def kernel(x_ggnn, x_appnp, edge_index, batch, W1, b1, gamma, beta,
           W2, b2, W3, b3):
    ne32 = _sc_edge_hist(edge_index, batch)
    batch3d = batch.reshape(_NB, 1, _BLK)

    blk_batch = pl.BlockSpec((1, 1, _BLK), lambda i: (i, 0, 0))
    blk_x = pl.BlockSpec((_BLK, _H), lambda i: (i, 0))
    full1 = lambda i: (0, 0)

    sums, cnt = pl.pallas_call(
        _sums_body,
        grid=(_NB,),
        in_specs=[blk_batch, blk_x, blk_x],
        out_specs=[pl.BlockSpec((_B, 2 * _H), full1),
                   pl.BlockSpec((_B, 1), full1)],
        out_shape=[jax.ShapeDtypeStruct((_B, 2 * _H), _F32),
                   jax.ShapeDtypeStruct((_B, 1), _F32)],
        scratch_shapes=[pltpu.VMEM((_B, 2 * _H), _F32),
                        pltpu.VMEM((_B, 1), _F32)],
    )(batch3d, x_ggnn, x_appnp)

    b1_2d = b1.reshape(1, _H)
    gamma_2d = gamma.reshape(1, _H)
    beta_2d = beta.reshape(1, _H)
    b2_2d = b2.reshape(1, _H // 2)
    W3l = W3[:, 0:1]
    W3g = W3[:, 1:2]
    b3l = b3[0:1].reshape(1, 1)
    b3g = b3[1:2].reshape(1, 1)

    nw = ne32.shape[0]
    out, wl, wg, cf = pl.pallas_call(
        _gate_fuse_body,
        grid=(_NB,),
        in_specs=[
            pl.BlockSpec((nw, _B), full1),
            pl.BlockSpec((_B, 2 * _H), full1),
            pl.BlockSpec((_B, 1), full1),
            blk_batch,
            blk_x,
            blk_x,
            pl.BlockSpec((2 * _H + 4, _H), full1),
            pl.BlockSpec((1, _H), full1),
            pl.BlockSpec((1, _H), full1),
            pl.BlockSpec((1, _H), full1),
            pl.BlockSpec((_H, _H // 2), full1),
            pl.BlockSpec((1, _H // 2), full1),
            pl.BlockSpec((_B, 1), full1),
            pl.BlockSpec((_B, 1), full1),
            pl.BlockSpec((1, 1), full1),
            pl.BlockSpec((1, 1), full1),
        ],
        out_specs=[
            pl.BlockSpec((_BLK, _H), lambda i: (i, 0)),
            pl.BlockSpec((_B, 1), full1),
            pl.BlockSpec((_B, 1), full1),
            pl.BlockSpec((_B, 4), full1),
        ],
        out_shape=[
            jax.ShapeDtypeStruct((_N, _H), _F32),
            jax.ShapeDtypeStruct((_B, 1), _F32),
            jax.ShapeDtypeStruct((_B, 1), _F32),
            jax.ShapeDtypeStruct((_B, 4), _F32),
        ],
        scratch_shapes=[pltpu.VMEM((_B, 1), _F32),
                        pltpu.VMEM((_B, 1), _F32)],
    )(ne32, sums, cnt, batch3d, x_ggnn, x_appnp, W1, b1_2d, gamma_2d,
      beta_2d, W2, b2_2d, W3l, W3g, b3l, b3g)

    return out, wl.reshape(_B), wg.reshape(_B), cf


# speedup vs baseline: 109.6527x; 1.1654x over previous
import functools
import math

import jax
import jax.numpy as jnp
from jax import lax
from jax.experimental import pallas as pl
from jax.experimental.pallas import tpu as pltpu
from jax.experimental.pallas import tpu_sc as plsc

_N = 10000
_E = 320000
_B = 64
_H = 128
_MAX_NODES = 500
_F32 = jnp.float32

_BLK = 2000
_NB = _N // _BLK

_DN_T = (((0,), (0,)), ((), ()))
_DN_M = (((1,), (0,)), ((), ()))


def _dot(a, b, dn):
    return jax.lax.dot_general(a, b, dn, preferred_element_type=_F32)


def _oh_mask(batch_ref):
    bat = batch_ref[...].reshape(1, _BLK)
    rows = jax.lax.broadcasted_iota(jnp.int32, (_B, 1), 0)
    return bat == rows


def _sums_body(batch_ref, xg_ref, xa_ref, sums_ref, cnt_ref, accx, acccnt):
    i = pl.program_id(0)
    same = _oh_mask(batch_ref)

    @pl.when(i == 0)
    def _init():
        accx[...] = jnp.zeros_like(accx)
        acccnt[...] = jnp.zeros_like(acccnt)

    ohb = same.astype(jnp.bfloat16)
    xcat = jnp.concatenate([xg_ref[...].astype(jnp.bfloat16),
                            xa_ref[...].astype(jnp.bfloat16)], axis=1)
    accx[...] += _dot(ohb, xcat, _DN_M)
    acccnt[...] += jnp.sum(same.astype(_F32), axis=1, keepdims=True)

    @pl.when(i == _NB - 1)
    def _emit():
        sums_ref[...] = accx[...]
        cnt_ref[...] = acccnt[...]


def _gate_fuse_body(ne_ref, sums_ref, cnt_ref, batch_ref, xg_ref, xa_ref,
                    W1_ref, b1_ref, g_ref, be_ref, W2_ref, b2_ref,
                    W3l_ref, W3g_ref, b3l_ref, b3g_ref,
                    out_ref, wl_ref, wg_ref, cf_ref, wls, wgs):
    i = pl.program_id(0)

    @pl.when(i == 0)
    def _mlp():
        counts = cnt_ref[...]
        pooled_g = sums_ref[:, 0:_H] / counts
        pooled_a = sums_ref[:, _H:2 * _H] / counts
        nw = ne_ref.shape[0]
        ne = _dot(ne_ref[...], jnp.full((nw, 1), 1.0, _F32), _DN_T)

        n_nodes = counts
        scale = jnp.log(n_nodes + 1.0) * (1.0 / math.log(_MAX_NODES + 1))
        density = ne / (n_nodes * (n_nodes - 1.0) + 1e-08)
        avg_degree = ne / (n_nodes + 1e-08)
        adn = jnp.minimum(avg_degree / 10.0, 1.0)

        num = jnp.sum(pooled_g * pooled_a, axis=1, keepdims=True)
        ngn = jnp.sqrt(jnp.sum(pooled_g * pooled_g, axis=1, keepdims=True))
        nan_ = jnp.sqrt(jnp.sum(pooled_a * pooled_a, axis=1, keepdims=True))
        cos = num / (jnp.maximum(ngn, 1e-08) * jnp.maximum(nan_, 1e-08))
        div = (1.0 - cos) * 0.5

        lane4 = jax.lax.broadcasted_iota(jnp.int32, (_B, 4), 1)
        cf = (jnp.where(lane4 == 0, scale, 0.0)
              + jnp.where(lane4 == 1, density, 0.0)
              + jnp.where(lane4 == 2, adn, 0.0)
              + jnp.where(lane4 == 3, div, 0.0))
        cf_ref[...] = cf

        h = (_dot(pooled_g, W1_ref[0:_H, :], _DN_M)
             + _dot(pooled_a, W1_ref[_H:2 * _H, :], _DN_M)
             + _dot(cf, W1_ref[2 * _H:2 * _H + 4, :], _DN_M)
             + b1_ref[...])
        mu = jnp.mean(h, axis=1, keepdims=True)
        var = jnp.mean((h - mu) ** 2, axis=1, keepdims=True)
        h = (h - mu) * jax.lax.rsqrt(var + 1e-05) * g_ref[...] + be_ref[...]
        h = jnp.maximum(h, 0.0)
        h2 = jnp.maximum(_dot(h, W2_ref[...], _DN_M) + b2_ref[...], 0.0)
        raw_l = _dot(h2, W3l_ref[...], _DN_M) + b3l_ref[...]
        raw_g = _dot(h2, W3g_ref[...], _DN_M) + b3g_ref[...]
        m = jnp.maximum(raw_l, raw_g)
        el = jnp.exp(raw_l - m)
        eg = jnp.exp(raw_g - m)
        s = el + eg
        wl = el / s
        wg = eg / s
        wl_ref[...] = wl.reshape(_B)
        wg_ref[...] = wg.reshape(_B)
        wls[...] = wl
        wgs[...] = wg

    ohf = _oh_mask(batch_ref).astype(_F32)
    wn_l = _dot(ohf, wls[...], _DN_T)
    wn_g = _dot(ohf, wgs[...], _DN_T)
    out_ref[...] = wn_l * xg_ref[...] + wn_g * xa_ref[...]


def _sc_edge_hist(edge_index, batch):
    info = plsc.get_sparse_core_info()
    nc, ns, nl = info.num_cores, info.num_subcores, info.num_lanes
    nw = nc * ns
    tiles = _E // 128
    tpw = tiles // nw
    cols = tpw * 128
    ntail = tiles - tpw * nw
    tail0 = tpw * nw * 128
    mesh = plsc.VectorSubcoreMesh(core_axis_name="c", subcore_axis_name="s")

    @functools.partial(
        pl.kernel, mesh=mesh,
        compiler_params=pltpu.CompilerParams(needs_layout_passes=False),
        out_type=jax.ShapeDtypeStruct((nw, _B), _F32),
        scratch_types=[
            pltpu.VMEM((_N,), jnp.int32),
            pltpu.VMEM((2, cols), jnp.int32),
            pltpu.VMEM((2, 128), jnp.int32),
            pltpu.VMEM((_B,), _F32),
            pltpu.VMEM((_B,), _F32),
        ],
    )
    def sc_kern(edge_hbm, batch_hbm, out_hbm, batch_v, ev_v, tail_v,
                cnt_v, cnt2_v):
        wid = lax.axis_index("s") * nc + lax.axis_index("c")
        pltpu.sync_copy(batch_hbm, batch_v)
        pltpu.sync_copy(edge_hbm.at[:, pl.ds(wid * cols, cols)], ev_v)
        zeros = jnp.zeros((nl,), _F32)
        for k in range(_B // nl):
            cnt_v[pl.ds(k * nl, nl)] = zeros
            cnt2_v[pl.ds(k * nl, nl)] = zeros
        ones = jnp.ones((nl,), _F32)
        unroll = 8
        step = unroll * nl

        def make_body(ev_ref):
            def body(j, carry):
                base_j = j * step
                gathered = []
                for u in range(unroll):
                    sl = ev_ref[0, pl.ds(base_j + u * nl, nl)]
                    dl = ev_ref[1, pl.ds(base_j + u * nl, nl)]
                    bs = plsc.load_gather(batch_v, [sl])
                    bd = plsc.load_gather(batch_v, [dl])
                    gathered.append((bs, bd))
                for u, (bs, bd) in enumerate(gathered):
                    tgt = cnt_v if u % 2 == 0 else cnt2_v
                    plsc.addupdate_scatter(tgt, [bs], ones, mask=bs == bd)
                return carry
            return body

        lax.fori_loop(0, cols // step, make_body(ev_v), 0)

        @pl.when(wid < ntail)
        def _tail():
            pltpu.sync_copy(edge_hbm.at[:, pl.ds(tail0 + wid * 128, 128)],
                            tail_v)
            lax.fori_loop(0, 1, make_body(tail_v), 0)

        for k in range(_B // nl):
            sl = pl.ds(k * nl, nl)
            cnt_v[sl] = cnt_v[sl] + cnt2_v[sl]
        pltpu.sync_copy(cnt_v, out_hbm.at[wid])

    return sc_kern(edge_index, batch)


def kernel(x_ggnn, x_appnp, edge_index, batch, W1, b1, gamma, beta,
           W2, b2, W3, b3):
    ne32 = _sc_edge_hist(edge_index, batch)
    batch3d = batch.reshape(_NB, 1, _BLK)

    blk_batch = pl.BlockSpec((1, 1, _BLK), lambda i: (i, 0, 0))
    blk_x = pl.BlockSpec((_BLK, _H), lambda i: (i, 0))
    full1 = lambda i: (0, 0)

    sums, cnt = pl.pallas_call(
        _sums_body,
        grid=(_NB,),
        in_specs=[blk_batch, blk_x, blk_x],
        out_specs=[pl.BlockSpec((_B, 2 * _H), full1),
                   pl.BlockSpec((_B, 1), full1)],
        out_shape=[jax.ShapeDtypeStruct((_B, 2 * _H), _F32),
                   jax.ShapeDtypeStruct((_B, 1), _F32)],
        scratch_shapes=[pltpu.VMEM((_B, 2 * _H), _F32),
                        pltpu.VMEM((_B, 1), _F32)],
    )(batch3d, x_ggnn, x_appnp)

    b1_2d = b1.reshape(1, _H)
    gamma_2d = gamma.reshape(1, _H)
    beta_2d = beta.reshape(1, _H)
    b2_2d = b2.reshape(1, _H // 2)
    W3l = W3[:, 0:1]
    W3g = W3[:, 1:2]
    b3l = b3[0:1].reshape(1, 1)
    b3g = b3[1:2].reshape(1, 1)

    nw = ne32.shape[0]
    out, wl, wg, cf = pl.pallas_call(
        _gate_fuse_body,
        grid=(_NB,),
        in_specs=[
            pl.BlockSpec((nw, _B), full1),
            pl.BlockSpec((_B, 2 * _H), full1),
            pl.BlockSpec((_B, 1), full1),
            blk_batch,
            blk_x,
            blk_x,
            pl.BlockSpec((2 * _H + 4, _H), full1),
            pl.BlockSpec((1, _H), full1),
            pl.BlockSpec((1, _H), full1),
            pl.BlockSpec((1, _H), full1),
            pl.BlockSpec((_H, _H // 2), full1),
            pl.BlockSpec((1, _H // 2), full1),
            pl.BlockSpec((_B, 1), full1),
            pl.BlockSpec((_B, 1), full1),
            pl.BlockSpec((1, 1), full1),
            pl.BlockSpec((1, 1), full1),
        ],
        out_specs=[
            pl.BlockSpec((_BLK, _H), lambda i: (i, 0)),
            pl.BlockSpec((_B,), lambda i: (0,)),
            pl.BlockSpec((_B,), lambda i: (0,)),
            pl.BlockSpec((_B, 4), full1),
        ],
        out_shape=[
            jax.ShapeDtypeStruct((_N, _H), _F32),
            jax.ShapeDtypeStruct((_B,), _F32),
            jax.ShapeDtypeStruct((_B,), _F32),
            jax.ShapeDtypeStruct((_B, 4), _F32),
        ],
        scratch_shapes=[pltpu.VMEM((_B, 1), _F32),
                        pltpu.VMEM((_B, 1), _F32)],
    )(ne32, sums, cnt, batch3d, x_ggnn, x_appnp, W1, b1_2d, gamma_2d,
      beta_2d, W2, b2_2d, W3l, W3g, b3l, b3g)

    return out, wl, wg, cf


# speedup vs baseline: 115.6697x; 1.0549x over previous
import functools
import math

import jax
import jax.numpy as jnp
from jax import lax
from jax.experimental import pallas as pl
from jax.experimental.pallas import tpu as pltpu
from jax.experimental.pallas import tpu_sc as plsc

_N = 10000
_E = 320000
_B = 64
_H = 128
_MAX_NODES = 500
_F32 = jnp.float32

_BLK = 2000
_NB = _N // _BLK

_DN_T = (((0,), (0,)), ((), ()))
_DN_M = (((1,), (0,)), ((), ()))


def _dot(a, b, dn):
    return jax.lax.dot_general(a, b, dn, preferred_element_type=_F32)


def _oh_mask(batch_ref):
    bat = batch_ref[...].reshape(1, _BLK)
    rows = jax.lax.broadcasted_iota(jnp.int32, (_B, 1), 0)
    return bat == rows


def _sums_body(batch_ref, xg_ref, xa_ref, sums_ref, cnt_ref, accx, acccnt):
    i = pl.program_id(0)
    same = _oh_mask(batch_ref)

    @pl.when(i == 0)
    def _init():
        accx[...] = jnp.zeros_like(accx)
        acccnt[...] = jnp.zeros_like(acccnt)

    ohb = same.astype(jnp.bfloat16)
    xcat = jnp.concatenate([xg_ref[...].astype(jnp.bfloat16),
                            xa_ref[...].astype(jnp.bfloat16)], axis=1)
    accx[...] += _dot(ohb, xcat, _DN_M)
    acccnt[...] += jnp.sum(same.astype(_F32), axis=1, keepdims=True)

    @pl.when(i == _NB - 1)
    def _emit():
        sums_ref[...] = accx[...]
        cnt_ref[...] = acccnt[...]


def _gate_fuse_body(ne_ref, sums_ref, cnt_ref, batch_ref, xg_ref, xa_ref,
                    W1_ref, b1_ref, g_ref, be_ref, W2_ref, b2_ref,
                    W3_ref, b3_ref,
                    out_ref, wl_ref, wg_ref, cf_ref, wls, wgs):
    i = pl.program_id(0)

    @pl.when(i == 0)
    def _mlp():
        counts = cnt_ref[...]
        pooled_g = sums_ref[:, 0:_H] / counts
        pooled_a = sums_ref[:, _H:2 * _H] / counts
        nw = ne_ref.shape[0]
        ne = _dot(ne_ref[...], jnp.full((nw, 1), 1.0, _F32), _DN_T)

        n_nodes = counts
        scale = jnp.log(n_nodes + 1.0) * (1.0 / math.log(_MAX_NODES + 1))
        density = ne / (n_nodes * (n_nodes - 1.0) + 1e-08)
        avg_degree = ne / (n_nodes + 1e-08)
        adn = jnp.minimum(avg_degree / 10.0, 1.0)

        num = jnp.sum(pooled_g * pooled_a, axis=1, keepdims=True)
        ngn = jnp.sqrt(jnp.sum(pooled_g * pooled_g, axis=1, keepdims=True))
        nan_ = jnp.sqrt(jnp.sum(pooled_a * pooled_a, axis=1, keepdims=True))
        cos = num / (jnp.maximum(ngn, 1e-08) * jnp.maximum(nan_, 1e-08))
        div = (1.0 - cos) * 0.5

        lane4 = jax.lax.broadcasted_iota(jnp.int32, (_B, 4), 1)
        cf = (jnp.where(lane4 == 0, scale, 0.0)
              + jnp.where(lane4 == 1, density, 0.0)
              + jnp.where(lane4 == 2, adn, 0.0)
              + jnp.where(lane4 == 3, div, 0.0))
        cf_ref[...] = cf

        h = (_dot(pooled_g, W1_ref[0:_H, :], _DN_M)
             + _dot(pooled_a, W1_ref[_H:2 * _H, :], _DN_M)
             + _dot(cf, W1_ref[2 * _H:2 * _H + 4, :], _DN_M)
             + b1_ref[...].reshape(1, _H))
        mu = jnp.mean(h, axis=1, keepdims=True)
        var = jnp.mean((h - mu) ** 2, axis=1, keepdims=True)
        h = ((h - mu) * jax.lax.rsqrt(var + 1e-05)
             * g_ref[...].reshape(1, _H) + be_ref[...].reshape(1, _H))
        h = jnp.maximum(h, 0.0)
        h2 = jnp.maximum(
            _dot(h, W2_ref[...], _DN_M) + b2_ref[...].reshape(1, _H // 2),
            0.0)
        raw = _dot(h2, W3_ref[...], _DN_M) + b3_ref[...].reshape(1, 2)
        raw_l = raw[:, 0:1]
        raw_g = raw[:, 1:2]
        m = jnp.maximum(raw_l, raw_g)
        el = jnp.exp(raw_l - m)
        eg = jnp.exp(raw_g - m)
        s = el + eg
        wl = el / s
        wg = eg / s
        wl_ref[...] = wl.reshape(_B)
        wg_ref[...] = wg.reshape(_B)
        wls[...] = wl
        wgs[...] = wg

    ohf = _oh_mask(batch_ref).astype(_F32)
    wn_l = _dot(ohf, wls[...], _DN_T)
    wn_g = _dot(ohf, wgs[...], _DN_T)
    out_ref[...] = wn_l * xg_ref[...] + wn_g * xa_ref[...]


def _sc_edge_hist(edge_index, batch):
    info = plsc.get_sparse_core_info()
    nc, ns, nl = info.num_cores, info.num_subcores, info.num_lanes
    nw = nc * ns
    tiles = _E // 128
    tpw = tiles // nw
    cols = tpw * 128
    ntail = tiles - tpw * nw
    tail0 = tpw * nw * 128
    mesh = plsc.VectorSubcoreMesh(core_axis_name="c", subcore_axis_name="s")

    @functools.partial(
        pl.kernel, mesh=mesh,
        compiler_params=pltpu.CompilerParams(needs_layout_passes=False),
        out_type=jax.ShapeDtypeStruct((nw, _B), _F32),
        scratch_types=[
            pltpu.VMEM((_N,), jnp.int32),
            pltpu.VMEM((2, cols), jnp.int32),
            pltpu.VMEM((2, 128), jnp.int32),
            pltpu.VMEM((_B,), _F32),
            pltpu.VMEM((_B,), _F32),
            pltpu.SemaphoreType.DMA,
            pltpu.SemaphoreType.DMA,
        ],
    )
    def sc_kern(edge_hbm, batch_hbm, out_hbm, batch_v, ev_v, tail_v,
                cnt_v, cnt2_v, sem_b, sem_e):
        wid = lax.axis_index("s") * nc + lax.axis_index("c")
        cp_b = pltpu.async_copy(batch_hbm, batch_v, sem_b)
        cp_e = pltpu.async_copy(edge_hbm.at[:, pl.ds(wid * cols, cols)],
                                ev_v, sem_e)
        cp_b.wait()
        cp_e.wait()
        zeros = jnp.zeros((nl,), _F32)
        for k in range(_B // nl):
            cnt_v[pl.ds(k * nl, nl)] = zeros
            cnt2_v[pl.ds(k * nl, nl)] = zeros
        ones = jnp.ones((nl,), _F32)
        unroll = 8
        step = unroll * nl

        def make_body(ev_ref):
            def body(j, carry):
                base_j = j * step
                gathered = []
                for u in range(unroll):
                    sl = ev_ref[0, pl.ds(base_j + u * nl, nl)]
                    dl = ev_ref[1, pl.ds(base_j + u * nl, nl)]
                    bs = plsc.load_gather(batch_v, [sl])
                    bd = plsc.load_gather(batch_v, [dl])
                    gathered.append((bs, bd))
                for u, (bs, bd) in enumerate(gathered):
                    tgt = cnt_v if u % 2 == 0 else cnt2_v
                    plsc.addupdate_scatter(tgt, [bs], ones, mask=bs == bd)
                return carry
            return body

        lax.fori_loop(0, cols // step, make_body(ev_v), 0)

        @pl.when(wid < ntail)
        def _tail():
            pltpu.sync_copy(edge_hbm.at[:, pl.ds(tail0 + wid * 128, 128)],
                            tail_v)
            lax.fori_loop(0, 1, make_body(tail_v), 0)

        for k in range(_B // nl):
            sl = pl.ds(k * nl, nl)
            cnt_v[sl] = cnt_v[sl] + cnt2_v[sl]
        pltpu.sync_copy(cnt_v, out_hbm.at[wid])

    return sc_kern(edge_index, batch)


def kernel(x_ggnn, x_appnp, edge_index, batch, W1, b1, gamma, beta,
           W2, b2, W3, b3):
    ne32 = _sc_edge_hist(edge_index, batch)
    batch3d = batch.reshape(_NB, 1, _BLK)

    blk_batch = pl.BlockSpec((1, 1, _BLK), lambda i: (i, 0, 0))
    blk_x = pl.BlockSpec((_BLK, _H), lambda i: (i, 0))
    full1 = lambda i: (0, 0)

    sums, cnt = pl.pallas_call(
        _sums_body,
        grid=(_NB,),
        in_specs=[blk_batch, blk_x, blk_x],
        out_specs=[pl.BlockSpec((_B, 2 * _H), full1),
                   pl.BlockSpec((_B, 1), full1)],
        out_shape=[jax.ShapeDtypeStruct((_B, 2 * _H), _F32),
                   jax.ShapeDtypeStruct((_B, 1), _F32)],
        scratch_shapes=[pltpu.VMEM((_B, 2 * _H), _F32),
                        pltpu.VMEM((_B, 1), _F32)],
    )(batch3d, x_ggnn, x_appnp)

    nw = ne32.shape[0]
    out, wl, wg, cf = pl.pallas_call(
        _gate_fuse_body,
        grid=(_NB,),
        in_specs=[
            pl.BlockSpec((nw, _B), full1),
            pl.BlockSpec((_B, 2 * _H), full1),
            pl.BlockSpec((_B, 1), full1),
            blk_batch,
            blk_x,
            blk_x,
            pl.BlockSpec((2 * _H + 4, _H), full1),
            pl.BlockSpec((_H,), lambda i: (0,)),
            pl.BlockSpec((_H,), lambda i: (0,)),
            pl.BlockSpec((_H,), lambda i: (0,)),
            pl.BlockSpec((_H, _H // 2), full1),
            pl.BlockSpec((_H // 2,), lambda i: (0,)),
            pl.BlockSpec((_B, 2), full1),
            pl.BlockSpec((2,), lambda i: (0,)),
        ],
        out_specs=[
            pl.BlockSpec((_BLK, _H), lambda i: (i, 0)),
            pl.BlockSpec((_B,), lambda i: (0,)),
            pl.BlockSpec((_B,), lambda i: (0,)),
            pl.BlockSpec((_B, 4), full1),
        ],
        out_shape=[
            jax.ShapeDtypeStruct((_N, _H), _F32),
            jax.ShapeDtypeStruct((_B,), _F32),
            jax.ShapeDtypeStruct((_B,), _F32),
            jax.ShapeDtypeStruct((_B, 4), _F32),
        ],
        scratch_shapes=[pltpu.VMEM((_B, 1), _F32),
                        pltpu.VMEM((_B, 1), _F32)],
    )(ne32, sums, cnt, batch3d, x_ggnn, x_appnp, W1, b1, gamma,
      beta, W2, b2, W3, b3)

    return out, wl, wg, cf


# speedup vs baseline: 118.9231x; 1.0281x over previous
import functools
import math

import jax
import jax.numpy as jnp
from jax import lax
from jax.experimental import pallas as pl
from jax.experimental.pallas import tpu as pltpu
from jax.experimental.pallas import tpu_sc as plsc

_N = 10000
_E = 320000
_B = 64
_H = 128
_MAX_NODES = 500
_F32 = jnp.float32

_BLK = 5000
_NB = _N // _BLK

_DN_T = (((0,), (0,)), ((), ()))
_DN_M = (((1,), (0,)), ((), ()))


def _dot(a, b, dn):
    return jax.lax.dot_general(a, b, dn, preferred_element_type=_F32)


def _oh_mask(batch_ref):
    bat = batch_ref[...].reshape(1, _BLK)
    rows = jax.lax.broadcasted_iota(jnp.int32, (_B, 1), 0)
    return bat == rows


def _sums_body(batch_ref, xg_ref, xa_ref, sums_ref, cnt_ref, accx, acccnt):
    i = pl.program_id(0)
    same = _oh_mask(batch_ref)

    @pl.when(i == 0)
    def _init():
        accx[...] = jnp.zeros_like(accx)
        acccnt[...] = jnp.zeros_like(acccnt)

    ohb = same.astype(jnp.bfloat16)
    xcat = jnp.concatenate([xg_ref[...].astype(jnp.bfloat16),
                            xa_ref[...].astype(jnp.bfloat16)], axis=1)
    accx[...] += _dot(ohb, xcat, _DN_M)
    acccnt[...] += jnp.sum(same.astype(_F32), axis=1, keepdims=True)

    @pl.when(i == _NB - 1)
    def _emit():
        sums_ref[...] = accx[...]
        cnt_ref[...] = acccnt[...]


def _gate_fuse_body(ne_ref, sums_ref, cnt_ref, batch_ref, xg_ref, xa_ref,
                    W1_ref, b1_ref, g_ref, be_ref, W2_ref, b2_ref,
                    W3_ref, b3_ref,
                    out_ref, wl_ref, wg_ref, cf_ref, wls, wgs):
    i = pl.program_id(0)

    @pl.when(i == 0)
    def _mlp():
        counts = cnt_ref[...]
        pooled_g = sums_ref[:, 0:_H] / counts
        pooled_a = sums_ref[:, _H:2 * _H] / counts
        nw = ne_ref.shape[0]
        ne = _dot(ne_ref[...], jnp.full((nw, 1), 1.0, _F32), _DN_T)

        n_nodes = counts
        scale = jnp.log(n_nodes + 1.0) * (1.0 / math.log(_MAX_NODES + 1))
        density = ne / (n_nodes * (n_nodes - 1.0) + 1e-08)
        avg_degree = ne / (n_nodes + 1e-08)
        adn = jnp.minimum(avg_degree / 10.0, 1.0)

        num = jnp.sum(pooled_g * pooled_a, axis=1, keepdims=True)
        ngn = jnp.sqrt(jnp.sum(pooled_g * pooled_g, axis=1, keepdims=True))
        nan_ = jnp.sqrt(jnp.sum(pooled_a * pooled_a, axis=1, keepdims=True))
        cos = num / (jnp.maximum(ngn, 1e-08) * jnp.maximum(nan_, 1e-08))
        div = (1.0 - cos) * 0.5

        lane4 = jax.lax.broadcasted_iota(jnp.int32, (_B, 4), 1)
        cf = (jnp.where(lane4 == 0, scale, 0.0)
              + jnp.where(lane4 == 1, density, 0.0)
              + jnp.where(lane4 == 2, adn, 0.0)
              + jnp.where(lane4 == 3, div, 0.0))
        cf_ref[...] = cf

        h = (_dot(pooled_g, W1_ref[0:_H, :], _DN_M)
             + _dot(pooled_a, W1_ref[_H:2 * _H, :], _DN_M)
             + _dot(cf, W1_ref[2 * _H:2 * _H + 4, :], _DN_M)
             + b1_ref[...].reshape(1, _H))
        mu = jnp.mean(h, axis=1, keepdims=True)
        var = jnp.mean((h - mu) ** 2, axis=1, keepdims=True)
        h = ((h - mu) * jax.lax.rsqrt(var + 1e-05)
             * g_ref[...].reshape(1, _H) + be_ref[...].reshape(1, _H))
        h = jnp.maximum(h, 0.0)
        h2 = jnp.maximum(
            _dot(h, W2_ref[...], _DN_M) + b2_ref[...].reshape(1, _H // 2),
            0.0)
        raw = _dot(h2, W3_ref[...], _DN_M) + b3_ref[...].reshape(1, 2)
        raw_l = raw[:, 0:1]
        raw_g = raw[:, 1:2]
        m = jnp.maximum(raw_l, raw_g)
        el = jnp.exp(raw_l - m)
        eg = jnp.exp(raw_g - m)
        s = el + eg
        wl = el / s
        wg = eg / s
        wl_ref[...] = wl.reshape(_B)
        wg_ref[...] = wg.reshape(_B)
        wls[...] = wl
        wgs[...] = wg

    ohf = _oh_mask(batch_ref).astype(_F32)
    wn_l = _dot(ohf, wls[...], _DN_T)
    wn_g = _dot(ohf, wgs[...], _DN_T)
    out_ref[...] = wn_l * xg_ref[...] + wn_g * xa_ref[...]


def _sc_edge_hist(edge_index, batch):
    info = plsc.get_sparse_core_info()
    nc, ns, nl = info.num_cores, info.num_subcores, info.num_lanes
    nw = nc * ns
    tiles = _E // 128
    tpw = tiles // nw
    cols = tpw * 128
    ntail = tiles - tpw * nw
    tail0 = tpw * nw * 128
    mesh = plsc.VectorSubcoreMesh(core_axis_name="c", subcore_axis_name="s")

    @functools.partial(
        pl.kernel, mesh=mesh,
        compiler_params=pltpu.CompilerParams(needs_layout_passes=False),
        out_type=jax.ShapeDtypeStruct((nw, _B), _F32),
        scratch_types=[
            pltpu.VMEM((_N,), jnp.int32),
            pltpu.VMEM((2, cols), jnp.int32),
            pltpu.VMEM((2, 128), jnp.int32),
            pltpu.VMEM((_B,), _F32),
            pltpu.VMEM((_B,), _F32),
            pltpu.SemaphoreType.DMA,
            pltpu.SemaphoreType.DMA,
        ],
    )
    def sc_kern(edge_hbm, batch_hbm, out_hbm, batch_v, ev_v, tail_v,
                cnt_v, cnt2_v, sem_b, sem_e):
        wid = lax.axis_index("s") * nc + lax.axis_index("c")
        cp_b = pltpu.async_copy(batch_hbm, batch_v, sem_b)
        cp_e = pltpu.async_copy(edge_hbm.at[:, pl.ds(wid * cols, cols)],
                                ev_v, sem_e)
        cp_b.wait()
        cp_e.wait()
        zeros = jnp.zeros((nl,), _F32)
        for k in range(_B // nl):
            cnt_v[pl.ds(k * nl, nl)] = zeros
            cnt2_v[pl.ds(k * nl, nl)] = zeros
        ones = jnp.ones((nl,), _F32)
        unroll = 8
        step = unroll * nl

        def make_body(ev_ref):
            def body(j, carry):
                base_j = j * step
                gathered = []
                for u in range(unroll):
                    sl = ev_ref[0, pl.ds(base_j + u * nl, nl)]
                    dl = ev_ref[1, pl.ds(base_j + u * nl, nl)]
                    bs = plsc.load_gather(batch_v, [sl])
                    bd = plsc.load_gather(batch_v, [dl])
                    gathered.append((bs, bd))
                for u, (bs, bd) in enumerate(gathered):
                    tgt = cnt_v if u % 2 == 0 else cnt2_v
                    plsc.addupdate_scatter(tgt, [bs], ones, mask=bs == bd)
                return carry
            return body

        lax.fori_loop(0, cols // step, make_body(ev_v), 0)

        @pl.when(wid < ntail)
        def _tail():
            pltpu.sync_copy(edge_hbm.at[:, pl.ds(tail0 + wid * 128, 128)],
                            tail_v)
            lax.fori_loop(0, 1, make_body(tail_v), 0)

        for k in range(_B // nl):
            sl = pl.ds(k * nl, nl)
            cnt_v[sl] = cnt_v[sl] + cnt2_v[sl]
        pltpu.sync_copy(cnt_v, out_hbm.at[wid])

    return sc_kern(edge_index, batch)


def kernel(x_ggnn, x_appnp, edge_index, batch, W1, b1, gamma, beta,
           W2, b2, W3, b3):
    ne32 = _sc_edge_hist(edge_index, batch)
    batch3d = batch.reshape(_NB, 1, _BLK)

    blk_batch = pl.BlockSpec((1, 1, _BLK), lambda i: (i, 0, 0))
    blk_x = pl.BlockSpec((_BLK, _H), lambda i: (i, 0))
    full1 = lambda i: (0, 0)

    sums, cnt = pl.pallas_call(
        _sums_body,
        grid=(_NB,),
        in_specs=[blk_batch, blk_x, blk_x],
        out_specs=[pl.BlockSpec((_B, 2 * _H), full1),
                   pl.BlockSpec((_B, 1), full1)],
        out_shape=[jax.ShapeDtypeStruct((_B, 2 * _H), _F32),
                   jax.ShapeDtypeStruct((_B, 1), _F32)],
        scratch_shapes=[pltpu.VMEM((_B, 2 * _H), _F32),
                        pltpu.VMEM((_B, 1), _F32)],
    )(batch3d, x_ggnn, x_appnp)

    nw = ne32.shape[0]
    out, wl, wg, cf = pl.pallas_call(
        _gate_fuse_body,
        grid=(_NB,),
        in_specs=[
            pl.BlockSpec((nw, _B), full1),
            pl.BlockSpec((_B, 2 * _H), full1),
            pl.BlockSpec((_B, 1), full1),
            blk_batch,
            blk_x,
            blk_x,
            pl.BlockSpec((2 * _H + 4, _H), full1),
            pl.BlockSpec((_H,), lambda i: (0,)),
            pl.BlockSpec((_H,), lambda i: (0,)),
            pl.BlockSpec((_H,), lambda i: (0,)),
            pl.BlockSpec((_H, _H // 2), full1),
            pl.BlockSpec((_H // 2,), lambda i: (0,)),
            pl.BlockSpec((_B, 2), full1),
            pl.BlockSpec((2,), lambda i: (0,)),
        ],
        out_specs=[
            pl.BlockSpec((_BLK, _H), lambda i: (i, 0)),
            pl.BlockSpec((_B,), lambda i: (0,)),
            pl.BlockSpec((_B,), lambda i: (0,)),
            pl.BlockSpec((_B, 4), full1),
        ],
        out_shape=[
            jax.ShapeDtypeStruct((_N, _H), _F32),
            jax.ShapeDtypeStruct((_B,), _F32),
            jax.ShapeDtypeStruct((_B,), _F32),
            jax.ShapeDtypeStruct((_B, 4), _F32),
        ],
        scratch_shapes=[pltpu.VMEM((_B, 1), _F32),
                        pltpu.VMEM((_B, 1), _F32)],
    )(ne32, sums, cnt, batch3d, x_ggnn, x_appnp, W1, b1, gamma,
      beta, W2, b2, W3, b3)

    return out, wl, wg, cf


# speedup vs baseline: 119.9409x; 1.0086x over previous
import functools
import math

import jax
import jax.numpy as jnp
from jax import lax
from jax.experimental import pallas as pl
from jax.experimental.pallas import tpu as pltpu
from jax.experimental.pallas import tpu_sc as plsc

_N = 10000
_E = 320000
_B = 64
_H = 128
_MAX_NODES = 500
_F32 = jnp.float32

_BLK = 5000
_NB = _N // _BLK

_DN_T = (((0,), (0,)), ((), ()))
_DN_M = (((1,), (0,)), ((), ()))


def _dot(a, b, dn):
    return jax.lax.dot_general(a, b, dn, preferred_element_type=_F32)


def _oh_mask(batch_ref):
    bat = batch_ref[...].reshape(1, _BLK)
    rows = jax.lax.broadcasted_iota(jnp.int32, (_B, 1), 0)
    return bat == rows


def _sums_body(batch_ref, xg_ref, xa_ref, sums_ref, cnt_ref, accx, acccnt):
    i = pl.program_id(0)
    same = _oh_mask(batch_ref)

    @pl.when(i == 0)
    def _init():
        accx[...] = jnp.zeros_like(accx)
        acccnt[...] = jnp.zeros_like(acccnt)

    ohb = same.astype(jnp.bfloat16)
    xcat = jnp.concatenate([xg_ref[...].astype(jnp.bfloat16),
                            xa_ref[...].astype(jnp.bfloat16)], axis=1)
    accx[...] += _dot(ohb, xcat, _DN_M)
    acccnt[...] += jnp.sum(same.astype(_F32), axis=1, keepdims=True)

    @pl.when(i == _NB - 1)
    def _emit():
        sums_ref[...] = accx[...]
        cnt_ref[...] = acccnt[...]


def _gate_fuse_body(ne_ref, sums_ref, cnt_ref, batch_ref, xg_ref, xa_ref,
                    W1_ref, b1_ref, g_ref, be_ref, W2_ref, b2_ref,
                    W3_ref, b3_ref,
                    out_ref, wl_ref, wg_ref, cf_ref, wls, wgs):
    i = pl.program_id(0)

    @pl.when(i == 0)
    def _mlp():
        counts = cnt_ref[...]
        pooled_g = sums_ref[:, 0:_H] / counts
        pooled_a = sums_ref[:, _H:2 * _H] / counts
        nw = ne_ref.shape[0]
        ne = _dot(ne_ref[...], jnp.full((nw, 1), 1.0, _F32), _DN_T)

        n_nodes = counts
        scale = jnp.log(n_nodes + 1.0) * (1.0 / math.log(_MAX_NODES + 1))
        density = ne / (n_nodes * (n_nodes - 1.0) + 1e-08)
        avg_degree = ne / (n_nodes + 1e-08)
        adn = jnp.minimum(avg_degree / 10.0, 1.0)

        num = jnp.sum(pooled_g * pooled_a, axis=1, keepdims=True)
        ngn = jnp.sqrt(jnp.sum(pooled_g * pooled_g, axis=1, keepdims=True))
        nan_ = jnp.sqrt(jnp.sum(pooled_a * pooled_a, axis=1, keepdims=True))
        cos = num / (jnp.maximum(ngn, 1e-08) * jnp.maximum(nan_, 1e-08))
        div = (1.0 - cos) * 0.5

        lane4 = jax.lax.broadcasted_iota(jnp.int32, (_B, 4), 1)
        cf = (jnp.where(lane4 == 0, scale, 0.0)
              + jnp.where(lane4 == 1, density, 0.0)
              + jnp.where(lane4 == 2, adn, 0.0)
              + jnp.where(lane4 == 3, div, 0.0))
        cf_ref[...] = cf

        h = (_dot(pooled_g, W1_ref[0:_H, :], _DN_M)
             + _dot(pooled_a, W1_ref[_H:2 * _H, :], _DN_M)
             + _dot(cf, W1_ref[2 * _H:2 * _H + 4, :], _DN_M)
             + b1_ref[...].reshape(1, _H))
        mu = jnp.mean(h, axis=1, keepdims=True)
        var = jnp.mean((h - mu) ** 2, axis=1, keepdims=True)
        h = ((h - mu) * jax.lax.rsqrt(var + 1e-05)
             * g_ref[...].reshape(1, _H) + be_ref[...].reshape(1, _H))
        h = jnp.maximum(h, 0.0)
        h2 = jnp.maximum(
            _dot(h, W2_ref[...], _DN_M) + b2_ref[...].reshape(1, _H // 2),
            0.0)
        raw = _dot(h2, W3_ref[...], _DN_M) + b3_ref[...].reshape(1, 2)
        raw_l = raw[:, 0:1]
        raw_g = raw[:, 1:2]
        m = jnp.maximum(raw_l, raw_g)
        el = jnp.exp(raw_l - m)
        eg = jnp.exp(raw_g - m)
        s = el + eg
        wl = el / s
        wg = eg / s
        wl_ref[...] = wl.reshape(_B)
        wg_ref[...] = wg.reshape(_B)
        wls[...] = wl
        wgs[...] = wg

    ohf = _oh_mask(batch_ref).astype(_F32)
    wn_l = _dot(ohf, wls[...], _DN_T)
    wn_g = _dot(ohf, wgs[...], _DN_T)
    out_ref[...] = wn_l * xg_ref[...] + wn_g * xa_ref[...]


def _sc_edge_hist(edge_index, batch):
    info = plsc.get_sparse_core_info()
    nc, ns, nl = info.num_cores, info.num_subcores, info.num_lanes
    nw = nc * ns
    tiles = _E // 128
    tpw = tiles // nw
    cols = tpw * 128
    ntail = tiles - tpw * nw
    tail0 = tpw * nw * 128
    mesh = plsc.VectorSubcoreMesh(core_axis_name="c", subcore_axis_name="s")

    @functools.partial(
        pl.kernel, mesh=mesh,
        compiler_params=pltpu.CompilerParams(needs_layout_passes=False),
        out_type=jax.ShapeDtypeStruct((nw, _B), _F32),
        scratch_types=[
            pltpu.VMEM((_N,), jnp.int32),
            pltpu.VMEM((2, cols), jnp.int32),
            pltpu.VMEM((2, 128), jnp.int32),
            pltpu.VMEM((_B,), _F32),
            pltpu.VMEM((_B,), _F32),
            pltpu.SemaphoreType.DMA,
            pltpu.SemaphoreType.DMA,
        ],
    )
    def sc_kern(edge_hbm, batch_hbm, out_hbm, batch_v, ev_v, tail_v,
                cnt_v, cnt2_v, sem_b, sem_e):
        wid = lax.axis_index("s") * nc + lax.axis_index("c")
        half = cols // 2
        base0 = wid * cols
        cp_b = pltpu.async_copy(batch_hbm, batch_v, sem_b)
        cp_e0 = pltpu.async_copy(edge_hbm.at[:, pl.ds(base0, half)],
                                 ev_v.at[:, pl.ds(0, half)], sem_e)
        cp_b.wait()
        cp_e0.wait()
        cp_e1 = pltpu.async_copy(edge_hbm.at[:, pl.ds(base0 + half, half)],
                                 ev_v.at[:, pl.ds(half, half)], sem_e)
        zeros = jnp.zeros((nl,), _F32)
        for k in range(_B // nl):
            cnt_v[pl.ds(k * nl, nl)] = zeros
            cnt2_v[pl.ds(k * nl, nl)] = zeros
        ones = jnp.ones((nl,), _F32)
        unroll = 8
        step = unroll * nl

        def make_body(ev_ref):
            def body(j, carry):
                base_j = j * step
                gathered = []
                for u in range(unroll):
                    sl = ev_ref[0, pl.ds(base_j + u * nl, nl)]
                    dl = ev_ref[1, pl.ds(base_j + u * nl, nl)]
                    bs = plsc.load_gather(batch_v, [sl])
                    bd = plsc.load_gather(batch_v, [dl])
                    gathered.append((bs, bd))
                for u, (bs, bd) in enumerate(gathered):
                    tgt = cnt_v if u % 2 == 0 else cnt2_v
                    plsc.addupdate_scatter(tgt, [bs], ones, mask=bs == bd)
                return carry
            return body

        lax.fori_loop(0, half // step, make_body(ev_v), 0)
        cp_e1.wait()
        lax.fori_loop(half // step, cols // step, make_body(ev_v), 0)

        @pl.when(wid < ntail)
        def _tail():
            pltpu.sync_copy(edge_hbm.at[:, pl.ds(tail0 + wid * 128, 128)],
                            tail_v)
            lax.fori_loop(0, 1, make_body(tail_v), 0)

        for k in range(_B // nl):
            sl = pl.ds(k * nl, nl)
            cnt_v[sl] = cnt_v[sl] + cnt2_v[sl]
        pltpu.sync_copy(cnt_v, out_hbm.at[wid])

    return sc_kern(edge_index, batch)


def kernel(x_ggnn, x_appnp, edge_index, batch, W1, b1, gamma, beta,
           W2, b2, W3, b3):
    ne32 = _sc_edge_hist(edge_index, batch)
    batch3d = batch.reshape(_NB, 1, _BLK)

    blk_batch = pl.BlockSpec((1, 1, _BLK), lambda i: (i, 0, 0))
    blk_x = pl.BlockSpec((_BLK, _H), lambda i: (i, 0))
    full1 = lambda i: (0, 0)

    sums, cnt = pl.pallas_call(
        _sums_body,
        grid=(_NB,),
        in_specs=[blk_batch, blk_x, blk_x],
        out_specs=[pl.BlockSpec((_B, 2 * _H), full1),
                   pl.BlockSpec((_B, 1), full1)],
        out_shape=[jax.ShapeDtypeStruct((_B, 2 * _H), _F32),
                   jax.ShapeDtypeStruct((_B, 1), _F32)],
        scratch_shapes=[pltpu.VMEM((_B, 2 * _H), _F32),
                        pltpu.VMEM((_B, 1), _F32)],
    )(batch3d, x_ggnn, x_appnp)

    nw = ne32.shape[0]
    out, wl, wg, cf = pl.pallas_call(
        _gate_fuse_body,
        grid=(_NB,),
        in_specs=[
            pl.BlockSpec((nw, _B), full1),
            pl.BlockSpec((_B, 2 * _H), full1),
            pl.BlockSpec((_B, 1), full1),
            blk_batch,
            blk_x,
            blk_x,
            pl.BlockSpec((2 * _H + 4, _H), full1),
            pl.BlockSpec((_H,), lambda i: (0,)),
            pl.BlockSpec((_H,), lambda i: (0,)),
            pl.BlockSpec((_H,), lambda i: (0,)),
            pl.BlockSpec((_H, _H // 2), full1),
            pl.BlockSpec((_H // 2,), lambda i: (0,)),
            pl.BlockSpec((_B, 2), full1),
            pl.BlockSpec((2,), lambda i: (0,)),
        ],
        out_specs=[
            pl.BlockSpec((_BLK, _H), lambda i: (i, 0)),
            pl.BlockSpec((_B,), lambda i: (0,)),
            pl.BlockSpec((_B,), lambda i: (0,)),
            pl.BlockSpec((_B, 4), full1),
        ],
        out_shape=[
            jax.ShapeDtypeStruct((_N, _H), _F32),
            jax.ShapeDtypeStruct((_B,), _F32),
            jax.ShapeDtypeStruct((_B,), _F32),
            jax.ShapeDtypeStruct((_B, 4), _F32),
        ],
        scratch_shapes=[pltpu.VMEM((_B, 1), _F32),
                        pltpu.VMEM((_B, 1), _F32)],
    )(ne32, sums, cnt, batch3d, x_ggnn, x_appnp, W1, b1, gamma,
      beta, W2, b2, W3, b3)

    return out, wl, wg, cf
